# Initial kernel scaffold; baseline (speedup 1.0000x reference)
import jax
import jax.numpy as jnp
from jax.experimental import pallas as pl


def kernel(x, edge_index, emb1, emb2, W1, b1, W2, b2, Wl, bl):
    raise NotImplementedError("write your pallas kernel here")


# speedup vs baseline: 2.6511x; 2.6511x over previous
import functools

import jax
import jax.numpy as jnp
from jax import lax
from jax.experimental import pallas as pl
from jax.experimental.pallas import tpu as pltpu
from jax.experimental.pallas import tpu_sc as plsc

f32 = jnp.float32
i32 = jnp.int32

N = 10000
D = 128
H = 64
NCLS = H // 2
NP = 10240
YR = NP * 8
E = 320000
NC, NS = 2, 16
NW = NC * NS
CH = 128
NCH = 79
EW = CH * NCH
EP = EW * NW
EPC = EP // CH
CNTN = 40960
CNT_PER_SUB = CNTN // NS
ROWS_PER_SUB = NP // NS

_mesh = plsc.VectorSubcoreMesh(core_axis_name="c", subcore_axis_name="s")
_params = pltpu.CompilerParams(needs_layout_passes=False,
                               use_tc_tiling_on_sc=False)


@functools.partial(
    pl.kernel,
    mesh=_mesh,
    compiler_params=_params,
    out_type=(
        jax.ShapeDtypeStruct((EPC, CH), i32),
        jax.ShapeDtypeStruct((EPC, CH), i32),
        jax.ShapeDtypeStruct((EPC, CH), i32),
        jax.ShapeDtypeStruct((EPC, CH), i32),
        jax.ShapeDtypeStruct((NC * CNTN,), f32),
        jax.ShapeDtypeStruct((NC * CNTN,), f32),
    ),
    scratch_types=(
        pltpu.VMEM((2 * (N + 8),), f32),
        pltpu.VMEM((2 * (N + 8),), f32),
        pltpu.VMEM((CH,), i32),
        pltpu.VMEM((CH,), i32),
        pltpu.VMEM((CH,), i32),
        pltpu.VMEM((CH,), i32),
        pltpu.VMEM((CH,), i32),
        pltpu.VMEM((CH,), i32),
        pltpu.VMEM((CH,), f32),
        pltpu.VMEM((CH,), f32),
        pltpu.VMEM_SHARED((CNTN,), f32),
        pltpu.VMEM_SHARED((CNTN,), f32),
        pltpu.SemaphoreType.DMA,
    ),
)
def _pass_a(srcp, dstp, e1f, e2f,
            seg1o, seg2o, g1o, g2o, c1o, c2o,
            e1v, e2v, srcv, dstv, s1v, s2v, g1v, g2v, onesv, zb, c1sh, c2sh,
            sem):
    cid = lax.axis_index("c")
    sid = lax.axis_index("s")
    wid = sid * NC + cid

    pltpu.sync_copy(e1f, e1v)
    pltpu.sync_copy(e2f, e2v)
    for i in range(CH // 16):
        onesv[pl.ds(i * 16, 16)] = jnp.full((16,), 1.0, f32)
        zb[pl.ds(i * 16, 16)] = jnp.zeros((16,), f32)

    def zinit(k, carry):
        zsl = pl.ds(sid * CNT_PER_SUB + k * CH, CH)
        pltpu.sync_copy(zb, c1sh.at[zsl])
        pltpu.sync_copy(zb, c2sh.at[zsl])
        return carry

    lax.fori_loop(0, CNT_PER_SUB // CH, zinit, 0)
    plsc.subcore_barrier()

    def chunk(c, carry):
        row = wid * NCH + c
        pltpu.sync_copy(srcp.at[row, :], srcv)
        pltpu.sync_copy(dstp.at[row, :], dstv)
        two = jnp.full((16,), 2, i32)
        one = jnp.full((16,), 1, i32)
        zero = jnp.zeros((16,), i32)
        for i in range(CH // 16):
            sl = pl.ds(i * 16, 16)
            sv = srcv[sl]
            dv = dstv[sl]
            e1sx = plsc.load_gather(e1v, [sv * 2])
            e1sy = plsc.load_gather(e1v, [sv * 2 + 1])
            e1dx = plsc.load_gather(e1v, [dv * 2])
            e1dy = plsc.load_gather(e1v, [dv * 2 + 1])
            e2sx = plsc.load_gather(e2v, [sv * 2])
            e2sy = plsc.load_gather(e2v, [sv * 2 + 1])
            e2dx = plsc.load_gather(e2v, [dv * 2])
            e2dy = plsc.load_gather(e2v, [dv * 2 + 1])
            r1 = (jnp.where(e1dx - e1sx >= 0.0, two, zero)
                  + jnp.where(e1dy - e1sy >= 0.0, one, zero))
            r2 = (jnp.where(e2dx - e2sx >= 0.0, two, zero)
                  + jnp.where(e2dy - e2sy >= 0.0, one, zero))
            s1v[sl] = r1 * N + dv
            s2v[sl] = r2 * N + dv
            g1v[sl] = sv * 8 + r1
            g2v[sl] = sv * 8 + 4 + r2
        pltpu.sync_copy(s1v, seg1o.at[row, :])
        pltpu.sync_copy(s2v, seg2o.at[row, :])
        pltpu.sync_copy(g1v, g1o.at[row, :])
        pltpu.sync_copy(g2v, g2o.at[row, :])
        pltpu.sync_copy(onesv, c1sh.at[s1v], add=True)
        pltpu.sync_copy(onesv, c2sh.at[s2v], add=True)
        return carry

    lax.fori_loop(0, NCH, chunk, 0)
    plsc.subcore_barrier()

    def readout(k, carry):
        isl = pl.ds(sid * CNT_PER_SUB + k * CH, CH)
        osl = pl.ds(cid * CNTN + sid * CNT_PER_SUB + k * CH, CH)
        pltpu.sync_copy(c1sh.at[isl], zb)
        pltpu.sync_copy(zb, c1o.at[osl])
        pltpu.sync_copy(c2sh.at[isl], zb)
        pltpu.sync_copy(zb, c2o.at[osl])
        return carry

    lax.fori_loop(0, CNT_PER_SUB // CH, readout, 0)


@functools.partial(
    pl.kernel,
    mesh=_mesh,
    compiler_params=_params,
    out_type=(
        jax.ShapeDtypeStruct((EPC, CH), f32),
        jax.ShapeDtypeStruct((EPC, CH), f32),
        jax.ShapeDtypeStruct((EPC, CH), f32),
    ),
    scratch_types=(
        pltpu.VMEM((CNTN,), f32),
        pltpu.VMEM((CNTN,), f32),
        pltpu.VMEM((CH,), i32),
        pltpu.VMEM((CH,), i32),
        pltpu.VMEM((CH,), f32),
        pltpu.VMEM((CH,), f32),
        pltpu.VMEM((CH,), f32),
        pltpu.SemaphoreType.DMA,
    ),
)
def _pass_a2(s1i, s2i, ic1, ic2,
             w1o, w2o, wso,
             ic1v, ic2v, s1v, s2v, w1v, w2v, wsv, sem):
    cid = lax.axis_index("c")
    sid = lax.axis_index("s")
    wid = sid * NC + cid

    pltpu.sync_copy(ic1, ic1v)
    pltpu.sync_copy(ic2, ic2v)

    def chunk(c, carry):
        row = wid * NCH + c
        pltpu.sync_copy(s1i.at[row, :], s1v)
        pltpu.sync_copy(s2i.at[row, :], s2v)
        for i in range(CH // 16):
            sl = pl.ds(i * 16, 16)
            w1 = plsc.load_gather(ic1v, [s1v[sl]])
            w2 = plsc.load_gather(ic2v, [s2v[sl]])
            w1v[sl] = w1
            w2v[sl] = w2
            wsv[sl] = w1 + w2
        pltpu.sync_copy(w1v, w1o.at[row, :])
        pltpu.sync_copy(w2v, w2o.at[row, :])
        pltpu.sync_copy(wsv, wso.at[row, :])
        return carry

    lax.fori_loop(0, NCH, chunk, 0)


@functools.partial(
    pl.kernel,
    mesh=_mesh,
    compiler_params=_params,
    out_type=jax.ShapeDtypeStruct((NC * NP, H), f32),
    scratch_types=(
        pltpu.VMEM((CH,), i32),
        pltpu.VMEM((CH,), i32),
        pltpu.VMEM((CH,), i32),
        pltpu.VMEM((CH,), f32),
        pltpu.VMEM((CH,), f32),
        pltpu.VMEM((CH, H), f32),
        pltpu.VMEM((CH, H), f32),
        pltpu.VMEM((CH, H), f32),
        pltpu.VMEM_SHARED((NP, H), f32),
        pltpu.SemaphoreType.DMA,
    ),
)
def _pass_b(g1i, g2i, dsti, w1i, w2i, y8, zrows,
            h1o,
            g1v, g2v, dstv, w1b, w2b, rows1, rows2, valb, h1sh, sem):
    cid = lax.axis_index("c")
    sid = lax.axis_index("s")
    wid = sid * NC + cid

    def zinit(k, carry):
        rsl = pl.ds(sid * ROWS_PER_SUB + k * CH, CH)
        pltpu.sync_copy(zrows.at[rsl, :], valb)
        pltpu.sync_copy(valb, h1sh.at[rsl, :])
        return carry

    lax.fori_loop(0, ROWS_PER_SUB // CH, zinit, 0)
    plsc.subcore_barrier()

    def chunk(c, carry):
        row = wid * NCH + c
        pltpu.sync_copy(g1i.at[row, :], g1v)
        pltpu.sync_copy(g2i.at[row, :], g2v)
        pltpu.sync_copy(dsti.at[row, :], dstv)
        pltpu.sync_copy(w1i.at[row, :], w1b)
        pltpu.sync_copy(w2i.at[row, :], w2b)
        pltpu.async_copy(y8.at[g1v], rows1, sem).wait()
        pltpu.async_copy(y8.at[g2v], rows2, sem).wait()

        def feat(f, cc):
            fv = jnp.full((16,), f, i32)
            for i in range(CH // 16):
                jv = lax.iota(i32, 16) + (i * 16)
                r1 = plsc.load_gather(rows1, [jv, fv])
                r2 = plsc.load_gather(rows2, [jv, fv])
                val = r1 * w1b[pl.ds(i * 16, 16)] + r2 * w2b[pl.ds(i * 16, 16)]
                plsc.store_scatter(valb, [jv, fv], val)
            return cc

        lax.fori_loop(0, H, feat, 0)
        pltpu.sync_copy(valb, h1sh.at[dstv], add=True)
        return carry

    lax.fori_loop(0, NCH, chunk, 0)
    plsc.subcore_barrier()

    def readout(k, carry):
        isl = pl.ds(sid * ROWS_PER_SUB + k * CH, CH)
        osl = pl.ds(cid * NP + sid * ROWS_PER_SUB + k * CH, CH)
        pltpu.sync_copy(h1sh.at[isl, :], rows1)
        pltpu.sync_copy(rows1, h1o.at[osl, :])
        return carry

    lax.fori_loop(0, ROWS_PER_SUB // CH, readout, 0)


@functools.partial(
    pl.kernel,
    mesh=_mesh,
    compiler_params=_params,
    out_type=jax.ShapeDtypeStruct((NC * NP, H), f32),
    scratch_types=(
        pltpu.VMEM((CH,), i32),
        pltpu.VMEM((CH,), i32),
        pltpu.VMEM((CH,), f32),
        pltpu.VMEM((CH, H), f32),
        pltpu.VMEM((CH, H), f32),
        pltpu.VMEM_SHARED((NP, H), f32),
        pltpu.SemaphoreType.DMA,
    ),
)
def _pass_c(srcp, dstp, wsi, z, zrows,
            h2o,
            srcv, dstv, wsv, rows, valb, h2sh, sem):
    cid = lax.axis_index("c")
    sid = lax.axis_index("s")
    wid = sid * NC + cid

    def zinit(k, carry):
        rsl = pl.ds(sid * ROWS_PER_SUB + k * CH, CH)
        pltpu.sync_copy(zrows.at[rsl, :], valb)
        pltpu.sync_copy(valb, h2sh.at[rsl, :])
        return carry

    lax.fori_loop(0, ROWS_PER_SUB // CH, zinit, 0)
    plsc.subcore_barrier()

    def chunk(c, carry):
        row = wid * NCH + c
        pltpu.sync_copy(srcp.at[row, :], srcv)
        pltpu.sync_copy(dstp.at[row, :], dstv)
        pltpu.sync_copy(wsi.at[row, :], wsv)
        pltpu.async_copy(z.at[srcv], rows, sem).wait()

        def feat(f, cc):
            fv = jnp.full((16,), f, i32)
            for i in range(CH // 16):
                jv = lax.iota(i32, 16) + (i * 16)
                r = plsc.load_gather(rows, [jv, fv])
                val = r * wsv[pl.ds(i * 16, 16)]
                plsc.store_scatter(valb, [jv, fv], val)
            return cc

        lax.fori_loop(0, H, feat, 0)
        pltpu.sync_copy(valb, h2sh.at[dstv], add=True)
        return carry

    lax.fori_loop(0, NCH, chunk, 0)
    plsc.subcore_barrier()

    def readout(k, carry):
        isl = pl.ds(sid * ROWS_PER_SUB + k * CH, CH)
        osl = pl.ds(cid * NP + sid * ROWS_PER_SUB + k * CH, CH)
        pltpu.sync_copy(h2sh.at[isl, :], rows)
        pltpu.sync_copy(rows, h2o.at[osl, :])
        return carry

    lax.fori_loop(0, ROWS_PER_SUB // CH, readout, 0)


def _t1_body(x_ref, w_ref, o_ref):
    o_ref[...] = jnp.dot(x_ref[...], w_ref[...], preferred_element_type=f32)


_t1 = pl.pallas_call(
    _t1_body,
    grid=(NP // 1024,),
    in_specs=[pl.BlockSpec((1024, D), lambda i: (i, 0)),
              pl.BlockSpec((D, 8 * H), lambda i: (0, 0))],
    out_specs=pl.BlockSpec((1024, 8 * H), lambda i: (i, 0)),
    out_shape=jax.ShapeDtypeStruct((NP, 8 * H), f32),
)


def _tinv_body(c1_ref, c2_ref, o1_ref, o2_ref):
    o1_ref[...] = 1.0 / jnp.maximum(
        jnp.sum(c1_ref[...], axis=0, keepdims=True), 1.0)
    o2_ref[...] = 1.0 / jnp.maximum(
        jnp.sum(c2_ref[...], axis=0, keepdims=True), 1.0)


_tinv = pl.pallas_call(
    _tinv_body,
    out_shape=(jax.ShapeDtypeStruct((1, CNTN), f32),
               jax.ShapeDtypeStruct((1, CNTN), f32)),
)


def _t2_body(a_ref, b_ref, b1_ref, w2_ref, o_ref):
    h = jnp.maximum(a_ref[...] + b_ref[...] + b1_ref[...], 0.0)
    o_ref[...] = jnp.dot(h, w2_ref[...], preferred_element_type=f32)


_t2 = pl.pallas_call(
    _t2_body,
    grid=(NP // 1024,),
    in_specs=[pl.BlockSpec((1024, H), lambda i: (i, 0)),
              pl.BlockSpec((1024, H), lambda i: (i, 0)),
              pl.BlockSpec((1, H), lambda i: (0, 0)),
              pl.BlockSpec((H, H), lambda i: (0, 0))],
    out_specs=pl.BlockSpec((1024, H), lambda i: (i, 0)),
    out_shape=jax.ShapeDtypeStruct((NP, H), f32),
)


def _t3_body(a_ref, b_ref, b2_ref, wl_ref, bl_ref, o_ref):
    h = a_ref[...] + b_ref[...] + b2_ref[...]
    lg = jnp.dot(h, wl_ref[...], preferred_element_type=f32) + bl_ref[...]
    m = jnp.max(lg, axis=-1, keepdims=True)
    lse = m + jnp.log(jnp.sum(jnp.exp(lg - m), axis=-1, keepdims=True))
    o_ref[...] = lg - lse


_t3 = pl.pallas_call(
    _t3_body,
    grid=(NP // 1024,),
    in_specs=[pl.BlockSpec((1024, H), lambda i: (i, 0)),
              pl.BlockSpec((1024, H), lambda i: (i, 0)),
              pl.BlockSpec((1, H), lambda i: (0, 0)),
              pl.BlockSpec((H, 128), lambda i: (0, 0)),
              pl.BlockSpec((1, 128), lambda i: (0, 0))],
    out_specs=pl.BlockSpec((1024, 128), lambda i: (i, 0)),
    out_shape=jax.ShapeDtypeStruct((NP, 128), f32),
)


def kernel(x, edge_index, emb1, emb2, W1, b1, W2, b2, Wl, bl):
    src = edge_index[0].astype(i32)
    dst = edge_index[1].astype(i32)
    srcp = jnp.pad(src, (0, EP - E), constant_values=N).reshape(EPC, CH)
    dstp = jnp.pad(dst, (0, EP - E), constant_values=N).reshape(EPC, CH)
    e1f = jnp.pad(emb1, ((0, 8), (0, 0))).reshape(-1)
    e2f = jnp.pad(emb2, ((0, 8), (0, 0))).reshape(-1)
    zrows = jnp.zeros((NP, H), f32)

    seg1, seg2, g1, g2, c1p, c2p = _pass_a(srcp, dstp, e1f, e2f)

    xp = jnp.pad(x, ((0, NP - N), (0, 0)))
    W1b = W1.reshape(8, D, H).transpose(1, 0, 2).reshape(D, 8 * H)
    y = _t1(xp, W1b)
    y8 = y.reshape(YR, H)

    ic1m, ic2m = _tinv(c1p.reshape(NC, CNTN), c2p.reshape(NC, CNTN))
    w1, w2, ws = _pass_a2(seg1, seg2, ic1m.reshape(CNTN), ic2m.reshape(CNTN))

    h1p = _pass_b(g1, g2, dstp, w1, w2, y8, zrows).reshape(NC, NP, H)
    z = _t2(h1p[0], h1p[1], b1.reshape(1, H), W2 * 0.125)

    h2p = _pass_c(srcp, dstp, ws, z, zrows).reshape(NC, NP, H)
    Wlp = jnp.pad(Wl, ((0, 0), (0, 128 - NCLS)))
    blp = jnp.pad(bl, (0, 128 - NCLS), constant_values=-1e30).reshape(1, 128)
    out = _t3(h2p[0], h2p[1], b2.reshape(1, H), Wlp, blp)
    return out[:N, :NCLS]


# speedup vs baseline: 5.5909x; 2.1089x over previous
import functools

import jax
import jax.numpy as jnp
from jax import lax
from jax.experimental import pallas as pl
from jax.experimental.pallas import tpu as pltpu
from jax.experimental.pallas import tpu_sc as plsc

f32 = jnp.float32
i32 = jnp.int32

N = 10000
D = 128
H = 64
NCLS = H // 2
NP = 10240
YR = NP * 8
E = 320000
NC, NS = 2, 16
NW = NC * NS
CH = 128
NCH = 79
EW = CH * NCH
EP = EW * NW
EPC = EP // CH
CNTN = 40960
CNT_PER_SUB = CNTN // NS
ROWS_PER_SUB = NP // NS

_mesh = plsc.VectorSubcoreMesh(core_axis_name="c", subcore_axis_name="s")
_params = pltpu.CompilerParams(needs_layout_passes=False,
                               use_tc_tiling_on_sc=False)


@functools.partial(
    pl.kernel,
    mesh=_mesh,
    compiler_params=_params,
    out_type=(
        jax.ShapeDtypeStruct((EPC, CH), i32),
        jax.ShapeDtypeStruct((EPC, CH), i32),
        jax.ShapeDtypeStruct((EPC, CH), i32),
        jax.ShapeDtypeStruct((EPC, CH), i32),
        jax.ShapeDtypeStruct((NC * CNTN,), f32),
        jax.ShapeDtypeStruct((NC * CNTN,), f32),
    ),
    scratch_types=(
        pltpu.VMEM((2 * (N + 8),), f32),
        pltpu.VMEM((2 * (N + 8),), f32),
        pltpu.VMEM((CH,), i32),
        pltpu.VMEM((CH,), i32),
        pltpu.VMEM((CH,), i32),
        pltpu.VMEM((CH,), i32),
        pltpu.VMEM((CH,), i32),
        pltpu.VMEM((CH,), i32),
        pltpu.VMEM((CH,), f32),
        pltpu.VMEM((CH,), f32),
        pltpu.VMEM_SHARED((CNTN,), f32),
        pltpu.VMEM_SHARED((CNTN,), f32),
        pltpu.SemaphoreType.DMA,
    ),
)
def _pass_a(srcp, dstp, e1f, e2f,
            seg1o, seg2o, g1o, g2o, c1o, c2o,
            e1v, e2v, srcv, dstv, s1v, s2v, g1v, g2v, onesv, zb, c1sh, c2sh,
            sem):
    cid = lax.axis_index("c")
    sid = lax.axis_index("s")
    wid = sid * NC + cid

    pltpu.sync_copy(e1f, e1v)
    pltpu.sync_copy(e2f, e2v)
    for i in range(CH // 16):
        onesv[pl.ds(i * 16, 16)] = jnp.full((16,), 1.0, f32)
        zb[pl.ds(i * 16, 16)] = jnp.zeros((16,), f32)

    def zinit(k, carry):
        zsl = pl.ds(sid * CNT_PER_SUB + k * CH, CH)
        pltpu.sync_copy(zb, c1sh.at[zsl])
        pltpu.sync_copy(zb, c2sh.at[zsl])
        return carry

    lax.fori_loop(0, CNT_PER_SUB // CH, zinit, 0)
    plsc.subcore_barrier()

    def chunk(c, carry):
        row = wid * NCH + c
        pltpu.sync_copy(srcp.at[row, :], srcv)
        pltpu.sync_copy(dstp.at[row, :], dstv)
        two = jnp.full((16,), 2, i32)
        one = jnp.full((16,), 1, i32)
        zero = jnp.zeros((16,), i32)
        for i in range(CH // 16):
            sl = pl.ds(i * 16, 16)
            sv = srcv[sl]
            dv = dstv[sl]
            e1sx = plsc.load_gather(e1v, [sv * 2])
            e1sy = plsc.load_gather(e1v, [sv * 2 + 1])
            e1dx = plsc.load_gather(e1v, [dv * 2])
            e1dy = plsc.load_gather(e1v, [dv * 2 + 1])
            e2sx = plsc.load_gather(e2v, [sv * 2])
            e2sy = plsc.load_gather(e2v, [sv * 2 + 1])
            e2dx = plsc.load_gather(e2v, [dv * 2])
            e2dy = plsc.load_gather(e2v, [dv * 2 + 1])
            r1 = (jnp.where(e1dx - e1sx >= 0.0, two, zero)
                  + jnp.where(e1dy - e1sy >= 0.0, one, zero))
            r2 = (jnp.where(e2dx - e2sx >= 0.0, two, zero)
                  + jnp.where(e2dy - e2sy >= 0.0, one, zero))
            s1v[sl] = r1 * N + dv
            s2v[sl] = r2 * N + dv
            g1v[sl] = sv * 8 + r1
            g2v[sl] = sv * 8 + 4 + r2
        pltpu.sync_copy(s1v, seg1o.at[row, :])
        pltpu.sync_copy(s2v, seg2o.at[row, :])
        pltpu.sync_copy(g1v, g1o.at[row, :])
        pltpu.sync_copy(g2v, g2o.at[row, :])
        pltpu.sync_copy(onesv, c1sh.at[s1v], add=True)
        pltpu.sync_copy(onesv, c2sh.at[s2v], add=True)
        return carry

    lax.fori_loop(0, NCH, chunk, 0)
    plsc.subcore_barrier()

    def readout(k, carry):
        isl = pl.ds(sid * CNT_PER_SUB + k * CH, CH)
        osl = pl.ds(cid * CNTN + sid * CNT_PER_SUB + k * CH, CH)
        pltpu.sync_copy(c1sh.at[isl], zb)
        pltpu.sync_copy(zb, c1o.at[osl])
        pltpu.sync_copy(c2sh.at[isl], zb)
        pltpu.sync_copy(zb, c2o.at[osl])
        return carry

    lax.fori_loop(0, CNT_PER_SUB // CH, readout, 0)


@functools.partial(
    pl.kernel,
    mesh=_mesh,
    compiler_params=_params,
    out_type=(
        jax.ShapeDtypeStruct((EPC, CH), f32),
        jax.ShapeDtypeStruct((EPC, CH), f32),
        jax.ShapeDtypeStruct((EPC, CH), f32),
    ),
    scratch_types=(
        pltpu.VMEM((CNTN,), f32),
        pltpu.VMEM((CNTN,), f32),
        pltpu.VMEM((CH,), i32),
        pltpu.VMEM((CH,), i32),
        pltpu.VMEM((CH,), f32),
        pltpu.VMEM((CH,), f32),
        pltpu.VMEM((CH,), f32),
        pltpu.SemaphoreType.DMA,
    ),
)
def _pass_a2(s1i, s2i, ic1, ic2,
             w1o, w2o, wso,
             ic1v, ic2v, s1v, s2v, w1v, w2v, wsv, sem):
    cid = lax.axis_index("c")
    sid = lax.axis_index("s")
    wid = sid * NC + cid

    pltpu.sync_copy(ic1, ic1v)
    pltpu.sync_copy(ic2, ic2v)

    def chunk(c, carry):
        row = wid * NCH + c
        pltpu.sync_copy(s1i.at[row, :], s1v)
        pltpu.sync_copy(s2i.at[row, :], s2v)
        for i in range(CH // 16):
            sl = pl.ds(i * 16, 16)
            w1 = plsc.load_gather(ic1v, [s1v[sl]])
            w2 = plsc.load_gather(ic2v, [s2v[sl]])
            w1v[sl] = w1
            w2v[sl] = w2
            wsv[sl] = w1 + w2
        pltpu.sync_copy(w1v, w1o.at[row, :])
        pltpu.sync_copy(w2v, w2o.at[row, :])
        pltpu.sync_copy(wsv, wso.at[row, :])
        return carry

    lax.fori_loop(0, NCH, chunk, 0)


@functools.partial(
    pl.kernel,
    mesh=_mesh,
    compiler_params=_params,
    out_type=jax.ShapeDtypeStruct((NC * NP, H), f32),
    scratch_types=(
        pltpu.VMEM((CH,), i32),
        pltpu.VMEM((CH,), i32),
        pltpu.VMEM((CH,), i32),
        pltpu.VMEM((CH,), f32),
        pltpu.VMEM((CH,), f32),
        pltpu.VMEM((CH, H), f32),
        pltpu.VMEM((CH, H), f32),
        pltpu.VMEM((CH, H), f32),
        pltpu.VMEM_SHARED((NP, H), f32),
        pltpu.SemaphoreType.DMA,
        pltpu.SemaphoreType.DMA,
    ),
)
def _pass_b(g1i, g2i, dsti, w1i, w2i, y8, zrows,
            h1o,
            g1v, g2v, dstv, w1b, w2b, rows1, rows2, valb, h1sh, sem, sem2):
    cid = lax.axis_index("c")
    sid = lax.axis_index("s")
    wid = sid * NC + cid

    def zinit(k, carry):
        rsl = pl.ds(sid * ROWS_PER_SUB + k * CH, CH)
        pltpu.sync_copy(zrows.at[rsl, :], valb)
        pltpu.sync_copy(valb, h1sh.at[rsl, :])
        return carry

    lax.fori_loop(0, ROWS_PER_SUB // CH, zinit, 0)
    plsc.subcore_barrier()

    def chunk(c, carry):
        row = wid * NCH + c
        pltpu.sync_copy(g1i.at[row, :], g1v)
        pltpu.sync_copy(g2i.at[row, :], g2v)
        pltpu.sync_copy(dsti.at[row, :], dstv)
        pltpu.sync_copy(w1i.at[row, :], w1b)
        pltpu.sync_copy(w2i.at[row, :], w2b)
        d1 = pltpu.async_copy(y8.at[g1v], rows1, sem)
        d2 = pltpu.async_copy(y8.at[g2v], rows2, sem2)
        d1.wait()
        d2.wait()

        def edge(j, cc):
            jv = jnp.full((16,), j, i32)
            bw1 = plsc.load_gather(w1b, [jv])
            bw2 = plsc.load_gather(w2b, [jv])
            for k in range(H // 16):
                sl = pl.ds(k * 16, 16)
                valb[j, sl] = rows1[j, sl] * bw1 + rows2[j, sl] * bw2
            return cc

        lax.fori_loop(0, CH, edge, 0)
        pltpu.sync_copy(valb, h1sh.at[dstv], add=True)
        return carry

    lax.fori_loop(0, NCH, chunk, 0)
    plsc.subcore_barrier()

    def readout(k, carry):
        isl = pl.ds(sid * ROWS_PER_SUB + k * CH, CH)
        osl = pl.ds(cid * NP + sid * ROWS_PER_SUB + k * CH, CH)
        pltpu.sync_copy(h1sh.at[isl, :], rows1)
        pltpu.sync_copy(rows1, h1o.at[osl, :])
        return carry

    lax.fori_loop(0, ROWS_PER_SUB // CH, readout, 0)


@functools.partial(
    pl.kernel,
    mesh=_mesh,
    compiler_params=_params,
    out_type=jax.ShapeDtypeStruct((NC * NP, H), f32),
    scratch_types=(
        pltpu.VMEM((CH,), i32),
        pltpu.VMEM((CH,), i32),
        pltpu.VMEM((CH,), f32),
        pltpu.VMEM((CH, H), f32),
        pltpu.VMEM((CH, H), f32),
        pltpu.VMEM_SHARED((NP, H), f32),
        pltpu.SemaphoreType.DMA,
    ),
)
def _pass_c(srcp, dstp, wsi, z, zrows,
            h2o,
            srcv, dstv, wsv, rows, valb, h2sh, sem):
    cid = lax.axis_index("c")
    sid = lax.axis_index("s")
    wid = sid * NC + cid

    def zinit(k, carry):
        rsl = pl.ds(sid * ROWS_PER_SUB + k * CH, CH)
        pltpu.sync_copy(zrows.at[rsl, :], valb)
        pltpu.sync_copy(valb, h2sh.at[rsl, :])
        return carry

    lax.fori_loop(0, ROWS_PER_SUB // CH, zinit, 0)
    plsc.subcore_barrier()

    def chunk(c, carry):
        row = wid * NCH + c
        pltpu.sync_copy(srcp.at[row, :], srcv)
        pltpu.sync_copy(dstp.at[row, :], dstv)
        pltpu.sync_copy(wsi.at[row, :], wsv)
        pltpu.async_copy(z.at[srcv], rows, sem).wait()

        def edge(j, cc):
            jv = jnp.full((16,), j, i32)
            bw = plsc.load_gather(wsv, [jv])
            for k in range(H // 16):
                sl = pl.ds(k * 16, 16)
                valb[j, sl] = rows[j, sl] * bw
            return cc

        lax.fori_loop(0, CH, edge, 0)
        pltpu.sync_copy(valb, h2sh.at[dstv], add=True)
        return carry

    lax.fori_loop(0, NCH, chunk, 0)
    plsc.subcore_barrier()

    def readout(k, carry):
        isl = pl.ds(sid * ROWS_PER_SUB + k * CH, CH)
        osl = pl.ds(cid * NP + sid * ROWS_PER_SUB + k * CH, CH)
        pltpu.sync_copy(h2sh.at[isl, :], rows)
        pltpu.sync_copy(rows, h2o.at[osl, :])
        return carry

    lax.fori_loop(0, ROWS_PER_SUB // CH, readout, 0)


def _t1_body(x_ref, w_ref, o_ref):
    o_ref[...] = jnp.dot(x_ref[...], w_ref[...], preferred_element_type=f32)


_t1 = pl.pallas_call(
    _t1_body,
    grid=(NP // 1024,),
    in_specs=[pl.BlockSpec((1024, D), lambda i: (i, 0)),
              pl.BlockSpec((D, 8 * H), lambda i: (0, 0))],
    out_specs=pl.BlockSpec((1024, 8 * H), lambda i: (i, 0)),
    out_shape=jax.ShapeDtypeStruct((NP, 8 * H), f32),
)


def _tinv_body(c1_ref, c2_ref, o1_ref, o2_ref):
    o1_ref[...] = 1.0 / jnp.maximum(
        jnp.sum(c1_ref[...], axis=0, keepdims=True), 1.0)
    o2_ref[...] = 1.0 / jnp.maximum(
        jnp.sum(c2_ref[...], axis=0, keepdims=True), 1.0)


_tinv = pl.pallas_call(
    _tinv_body,
    out_shape=(jax.ShapeDtypeStruct((1, CNTN), f32),
               jax.ShapeDtypeStruct((1, CNTN), f32)),
)


def _t2_body(a_ref, b_ref, b1_ref, w2_ref, o_ref):
    h = jnp.maximum(a_ref[...] + b_ref[...] + b1_ref[...], 0.0)
    o_ref[...] = jnp.dot(h, w2_ref[...], preferred_element_type=f32)


_t2 = pl.pallas_call(
    _t2_body,
    grid=(NP // 1024,),
    in_specs=[pl.BlockSpec((1024, H), lambda i: (i, 0)),
              pl.BlockSpec((1024, H), lambda i: (i, 0)),
              pl.BlockSpec((1, H), lambda i: (0, 0)),
              pl.BlockSpec((H, H), lambda i: (0, 0))],
    out_specs=pl.BlockSpec((1024, H), lambda i: (i, 0)),
    out_shape=jax.ShapeDtypeStruct((NP, H), f32),
)


def _t3_body(a_ref, b_ref, b2_ref, wl_ref, bl_ref, o_ref):
    h = a_ref[...] + b_ref[...] + b2_ref[...]
    lg = jnp.dot(h, wl_ref[...], preferred_element_type=f32) + bl_ref[...]
    m = jnp.max(lg, axis=-1, keepdims=True)
    lse = m + jnp.log(jnp.sum(jnp.exp(lg - m), axis=-1, keepdims=True))
    o_ref[...] = lg - lse


_t3 = pl.pallas_call(
    _t3_body,
    grid=(NP // 1024,),
    in_specs=[pl.BlockSpec((1024, H), lambda i: (i, 0)),
              pl.BlockSpec((1024, H), lambda i: (i, 0)),
              pl.BlockSpec((1, H), lambda i: (0, 0)),
              pl.BlockSpec((H, 128), lambda i: (0, 0)),
              pl.BlockSpec((1, 128), lambda i: (0, 0))],
    out_specs=pl.BlockSpec((1024, 128), lambda i: (i, 0)),
    out_shape=jax.ShapeDtypeStruct((NP, 128), f32),
)


def kernel(x, edge_index, emb1, emb2, W1, b1, W2, b2, Wl, bl):
    src = edge_index[0].astype(i32)
    dst = edge_index[1].astype(i32)
    srcp = jnp.pad(src, (0, EP - E), constant_values=N).reshape(EPC, CH)
    dstp = jnp.pad(dst, (0, EP - E), constant_values=N).reshape(EPC, CH)
    e1f = jnp.pad(emb1, ((0, 8), (0, 0))).reshape(-1)
    e2f = jnp.pad(emb2, ((0, 8), (0, 0))).reshape(-1)
    zrows = jnp.zeros((NP, H), f32)

    seg1, seg2, g1, g2, c1p, c2p = _pass_a(srcp, dstp, e1f, e2f)

    xp = jnp.pad(x, ((0, NP - N), (0, 0)))
    W1b = W1.reshape(8, D, H).transpose(1, 0, 2).reshape(D, 8 * H)
    y = _t1(xp, W1b)
    y8 = y.reshape(YR, H)

    ic1m, ic2m = _tinv(c1p.reshape(NC, CNTN), c2p.reshape(NC, CNTN))
    w1, w2, ws = _pass_a2(seg1, seg2, ic1m.reshape(CNTN), ic2m.reshape(CNTN))

    h1p = _pass_b(g1, g2, dstp, w1, w2, y8, zrows).reshape(NC, NP, H)
    z = _t2(h1p[0], h1p[1], b1.reshape(1, H), W2 * 0.125)

    h2p = _pass_c(srcp, dstp, ws, z, zrows).reshape(NC, NP, H)
    Wlp = jnp.pad(Wl, ((0, 0), (0, 128 - NCLS)))
    blp = jnp.pad(bl, (0, 128 - NCLS), constant_values=-1e30).reshape(1, 128)
    out = _t3(h2p[0], h2p[1], b2.reshape(1, H), Wlp, blp)
    return out[:N, :NCLS]


# speedup vs baseline: 8.0584x; 1.4413x over previous
import functools

import jax
import jax.numpy as jnp
from jax import lax
from jax.experimental import pallas as pl
from jax.experimental.pallas import tpu as pltpu
from jax.experimental.pallas import tpu_sc as plsc

f32 = jnp.float32
i32 = jnp.int32

N = 10000
D = 128
H = 64
NCLS = H // 2
NP = 10240
YR = NP * 8
E = 320000
NC, NS = 2, 16
NW = NC * NS
CH = 128
NCH = 80
EW = CH * NCH
EP = EW * NW
EPC = EP // CH
CNTN = 40960
CNT_PER_SUB = CNTN // NS
ROWS_PER_SUB = NP // NS

_mesh = plsc.VectorSubcoreMesh(core_axis_name="c", subcore_axis_name="s")
_params = pltpu.CompilerParams(needs_layout_passes=False,
                               use_tc_tiling_on_sc=False)


@functools.partial(
    pl.kernel,
    mesh=_mesh,
    compiler_params=_params,
    out_type=(
        jax.ShapeDtypeStruct((EPC, CH), i32),
        jax.ShapeDtypeStruct((EPC, CH), i32),
        jax.ShapeDtypeStruct((EPC, 2 * CH), i32),
        jax.ShapeDtypeStruct((NC * CNTN,), f32),
        jax.ShapeDtypeStruct((NC * CNTN,), f32),
    ),
    scratch_types=(
        pltpu.VMEM((2 * (N + 8),), f32),
        pltpu.VMEM((2 * (N + 8),), f32),
        pltpu.VMEM((CH,), i32),
        pltpu.VMEM((CH,), i32),
        pltpu.VMEM((CH,), i32),
        pltpu.VMEM((CH,), i32),
        pltpu.VMEM((CH,), i32),
        pltpu.VMEM((CH,), i32),
        pltpu.VMEM((CH,), f32),
        pltpu.VMEM((CH,), f32),
        pltpu.VMEM_SHARED((CNTN,), f32),
        pltpu.VMEM_SHARED((CNTN,), f32),
        pltpu.SemaphoreType.DMA,
    ),
)
def _pass_a(srcp, dstp, e1f, e2f,
            seg1o, seg2o, gpo, c1o, c2o,
            e1v, e2v, srcv, dstv, s1v, s2v, g1v, g2v, onesv, zb, c1sh, c2sh,
            sem):
    cid = lax.axis_index("c")
    sid = lax.axis_index("s")
    wid = sid * NC + cid

    pltpu.sync_copy(e1f, e1v)
    pltpu.sync_copy(e2f, e2v)
    for i in range(CH // 16):
        onesv[pl.ds(i * 16, 16)] = jnp.full((16,), 1.0, f32)
        zb[pl.ds(i * 16, 16)] = jnp.zeros((16,), f32)

    def zinit(k, carry):
        zsl = pl.ds(sid * CNT_PER_SUB + k * CH, CH)
        pltpu.sync_copy(zb, c1sh.at[zsl])
        pltpu.sync_copy(zb, c2sh.at[zsl])
        return carry

    lax.fori_loop(0, CNT_PER_SUB // CH, zinit, 0)
    plsc.subcore_barrier()

    def chunk(c, carry):
        row = wid * NCH + c
        pltpu.sync_copy(srcp.at[row, :], srcv)
        pltpu.sync_copy(dstp.at[row, :], dstv)
        two = jnp.full((16,), 2, i32)
        one = jnp.full((16,), 1, i32)
        zero = jnp.zeros((16,), i32)
        for i in range(CH // 16):
            sl = pl.ds(i * 16, 16)
            sv = srcv[sl]
            dv = dstv[sl]
            e1sx = plsc.load_gather(e1v, [sv * 2])
            e1sy = plsc.load_gather(e1v, [sv * 2 + 1])
            e1dx = plsc.load_gather(e1v, [dv * 2])
            e1dy = plsc.load_gather(e1v, [dv * 2 + 1])
            e2sx = plsc.load_gather(e2v, [sv * 2])
            e2sy = plsc.load_gather(e2v, [sv * 2 + 1])
            e2dx = plsc.load_gather(e2v, [dv * 2])
            e2dy = plsc.load_gather(e2v, [dv * 2 + 1])
            r1 = (jnp.where(e1dx - e1sx >= 0.0, two, zero)
                  + jnp.where(e1dy - e1sy >= 0.0, one, zero))
            r2 = (jnp.where(e2dx - e2sx >= 0.0, two, zero)
                  + jnp.where(e2dy - e2sy >= 0.0, one, zero))
            s1v[sl] = r1 * N + dv
            s2v[sl] = r2 * N + dv
            g1v[sl] = sv * 8 + r1
            g2v[sl] = sv * 8 + 4 + r2
        pltpu.sync_copy(s1v, seg1o.at[row, :])
        pltpu.sync_copy(s2v, seg2o.at[row, :])
        pltpu.sync_copy(g1v, gpo.at[row, pl.ds(0, CH)])
        pltpu.sync_copy(g2v, gpo.at[row, pl.ds(CH, CH)])
        pltpu.sync_copy(onesv, c1sh.at[s1v], add=True)
        pltpu.sync_copy(onesv, c2sh.at[s2v], add=True)
        return carry

    lax.fori_loop(0, NCH, chunk, 0)
    plsc.subcore_barrier()

    def readout(k, carry):
        isl = pl.ds(sid * CNT_PER_SUB + k * CH, CH)
        osl = pl.ds(cid * CNTN + sid * CNT_PER_SUB + k * CH, CH)
        pltpu.sync_copy(c1sh.at[isl], zb)
        pltpu.sync_copy(zb, c1o.at[osl])
        pltpu.sync_copy(c2sh.at[isl], zb)
        pltpu.sync_copy(zb, c2o.at[osl])
        return carry

    lax.fori_loop(0, CNT_PER_SUB // CH, readout, 0)


@functools.partial(
    pl.kernel,
    mesh=_mesh,
    compiler_params=_params,
    out_type=(
        jax.ShapeDtypeStruct((EPC, 2 * CH), f32),
        jax.ShapeDtypeStruct((EPC, CH), f32),
    ),
    scratch_types=(
        pltpu.VMEM((CNTN,), f32),
        pltpu.VMEM((CNTN,), f32),
        pltpu.VMEM((CH,), i32),
        pltpu.VMEM((CH,), i32),
        pltpu.VMEM((CH,), f32),
        pltpu.VMEM((CH,), f32),
        pltpu.VMEM((CH,), f32),
        pltpu.SemaphoreType.DMA,
    ),
)
def _pass_a2(s1i, s2i, ic1, ic2,
             wpo, wso,
             ic1v, ic2v, s1v, s2v, w1v, w2v, wsv, sem):
    cid = lax.axis_index("c")
    sid = lax.axis_index("s")
    wid = sid * NC + cid

    pltpu.sync_copy(ic1, ic1v)
    pltpu.sync_copy(ic2, ic2v)

    def chunk(c, carry):
        row = wid * NCH + c
        pltpu.sync_copy(s1i.at[row, :], s1v)
        pltpu.sync_copy(s2i.at[row, :], s2v)
        for i in range(CH // 16):
            sl = pl.ds(i * 16, 16)
            w1 = plsc.load_gather(ic1v, [s1v[sl]])
            w2 = plsc.load_gather(ic2v, [s2v[sl]])
            w1v[sl] = w1
            w2v[sl] = w2
            wsv[sl] = w1 + w2
        pltpu.sync_copy(w1v, wpo.at[row, pl.ds(0, CH)])
        pltpu.sync_copy(w2v, wpo.at[row, pl.ds(CH, CH)])
        pltpu.sync_copy(wsv, wso.at[row, :])
        return carry

    lax.fori_loop(0, NCH, chunk, 0)


@functools.partial(
    pl.kernel,
    mesh=_mesh,
    compiler_params=_params,
    out_type=jax.ShapeDtypeStruct((NC * NP, H), f32),
    scratch_types=(
        pltpu.VMEM((2 * CH,), i32),
        pltpu.VMEM((2 * CH,), i32),
        pltpu.VMEM((CH,), i32),
        pltpu.VMEM((CH,), i32),
        pltpu.VMEM((2 * CH,), f32),
        pltpu.VMEM((2 * CH,), f32),
        pltpu.VMEM((CH,), i32),
        pltpu.VMEM((CH,), i32),
        pltpu.VMEM((2 * CH, H), f32),
        pltpu.VMEM((2 * CH, H), f32),
        pltpu.VMEM((CH, H), f32),
        pltpu.VMEM((CH, H), f32),
        pltpu.VMEM_SHARED((NP, H), f32),
        pltpu.SemaphoreType.DMA,
        pltpu.SemaphoreType.DMA,
        pltpu.SemaphoreType.DMA,
        pltpu.SemaphoreType.DMA,
        pltpu.SemaphoreType.DMA,
        pltpu.SemaphoreType.DMA,
    ),
)
def _pass_b(gpi, dsti, wpi, y8, zrows,
            h1o,
            gp0, gp1, dst0, dst1, wp0, wp1, ds0, ds1, rw0, rw1, vb0, vb1,
            h1sh, semi0, semi1, semg0, semg1, sems0, sems1):
    cid = lax.axis_index("c")
    sid = lax.axis_index("s")
    wid = sid * NC + cid

    gp = (gp0, gp1)
    dstb = (dst0, dst1)
    wp = (wp0, wp1)
    dss = (ds0, ds1)
    rw = (rw0, rw1)
    vb = (vb0, vb1)
    semi = (semi0, semi1)
    semg = (semg0, semg1)
    sems = (sems0, sems1)

    def zinit(k, carry):
        rsl = pl.ds(sid * ROWS_PER_SUB + k * CH, CH)
        pltpu.sync_copy(zrows.at[rsl, :], vb0)
        pltpu.sync_copy(vb0, h1sh.at[rsl, :])
        return carry

    lax.fori_loop(0, ROWS_PER_SUB // CH, zinit, 0)
    plsc.subcore_barrier()

    def issue_idx(c, b):
        row = wid * NCH + c
        pltpu.async_copy(gpi.at[row, :], gp[b], semi[b])
        pltpu.async_copy(dsti.at[row, :], dstb[b], semi[b])
        pltpu.async_copy(wpi.at[row, :], wp[b], semi[b])

    def wait_idx(c, b):
        row = wid * NCH + c
        pltpu.make_async_copy(gpi.at[row, :], gp[b], semi[b]).wait()
        pltpu.make_async_copy(dsti.at[row, :], dstb[b], semi[b]).wait()
        pltpu.make_async_copy(wpi.at[row, :], wp[b], semi[b]).wait()

    def issue_gather(b):
        pltpu.async_copy(y8.at[gp[b]], rw[b], semg[b])

    def wait_gather(b):
        pltpu.make_async_copy(y8.at[gp[b]], rw[b], semg[b]).wait()

    def issue_scatter(b):
        pltpu.async_copy(vb[b], h1sh.at[dss[b]], sems[b], add=True)

    def wait_scatter(b):
        pltpu.make_async_copy(vb[b], h1sh.at[dss[b]], sems[b]).wait()

    def compute(b):
        for i in range(CH // 16):
            sl = pl.ds(i * 16, 16)
            dss[b][sl] = dstb[b][sl]

        def edge(p, cc):
            for u in range(4):
                j = p * 4 + u
                jv = jnp.full((16,), j, i32)
                bw1 = plsc.load_gather(wp[b], [jv])
                bw2 = plsc.load_gather(wp[b], [jv + CH])
                for k in range(H // 16):
                    sl = pl.ds(k * 16, 16)
                    vb[b][j, sl] = (rw[b][j, sl] * bw1
                                    + rw[b][j + CH, sl] * bw2)
            return cc

        lax.fori_loop(0, CH // 4, edge, 0)

    issue_idx(0, 0)
    wait_idx(0, 0)
    issue_gather(0)
    issue_idx(1, 1)

    def pair(p, carry):
        for b in (0, 1):
            c = 2 * p + b
            bn = 1 - b
            wait_gather(b)

            @pl.when(c + 1 < NCH)
            def _():
                wait_idx(c + 1, bn)
                issue_gather(bn)

            @pl.when(c >= 2)
            def _():
                wait_scatter(b)

            compute(b)
            issue_scatter(b)

            @pl.when(c + 2 < NCH)
            def _():
                issue_idx(c + 2, b)
        return carry

    lax.fori_loop(0, NCH // 2, pair, 0)
    wait_scatter(0)
    wait_scatter(1)
    plsc.subcore_barrier()

    def readout(k, carry):
        isl = pl.ds(sid * ROWS_PER_SUB + k * CH, CH)
        osl = pl.ds(cid * NP + sid * ROWS_PER_SUB + k * CH, CH)
        pltpu.sync_copy(h1sh.at[isl, :], vb0)
        pltpu.sync_copy(vb0, h1o.at[osl, :])
        return carry

    lax.fori_loop(0, ROWS_PER_SUB // CH, readout, 0)


@functools.partial(
    pl.kernel,
    mesh=_mesh,
    compiler_params=_params,
    out_type=jax.ShapeDtypeStruct((NC * NP, H), f32),
    scratch_types=(
        pltpu.VMEM((CH,), i32),
        pltpu.VMEM((CH,), i32),
        pltpu.VMEM((CH,), i32),
        pltpu.VMEM((CH,), i32),
        pltpu.VMEM((CH,), f32),
        pltpu.VMEM((CH,), f32),
        pltpu.VMEM((CH,), i32),
        pltpu.VMEM((CH,), i32),
        pltpu.VMEM((CH, H), f32),
        pltpu.VMEM((CH, H), f32),
        pltpu.VMEM((CH, H), f32),
        pltpu.VMEM((CH, H), f32),
        pltpu.VMEM_SHARED((NP, H), f32),
        pltpu.SemaphoreType.DMA,
        pltpu.SemaphoreType.DMA,
        pltpu.SemaphoreType.DMA,
        pltpu.SemaphoreType.DMA,
        pltpu.SemaphoreType.DMA,
        pltpu.SemaphoreType.DMA,
    ),
)
def _pass_c(srcp, dstp, wsi, z, zrows,
            h2o,
            src0, src1, dst0, dst1, ws0, ws1, ds0, ds1, rw0, rw1, vb0, vb1,
            h2sh, semi0, semi1, semg0, semg1, sems0, sems1):
    cid = lax.axis_index("c")
    sid = lax.axis_index("s")
    wid = sid * NC + cid

    srcb = (src0, src1)
    dstb = (dst0, dst1)
    wsb = (ws0, ws1)
    dss = (ds0, ds1)
    rw = (rw0, rw1)
    vb = (vb0, vb1)
    semi = (semi0, semi1)
    semg = (semg0, semg1)
    sems = (sems0, sems1)

    def zinit(k, carry):
        rsl = pl.ds(sid * ROWS_PER_SUB + k * CH, CH)
        pltpu.sync_copy(zrows.at[rsl, :], vb0)
        pltpu.sync_copy(vb0, h2sh.at[rsl, :])
        return carry

    lax.fori_loop(0, ROWS_PER_SUB // CH, zinit, 0)
    plsc.subcore_barrier()

    def issue_idx(c, b):
        row = wid * NCH + c
        pltpu.async_copy(srcp.at[row, :], srcb[b], semi[b])
        pltpu.async_copy(dstp.at[row, :], dstb[b], semi[b])
        pltpu.async_copy(wsi.at[row, :], wsb[b], semi[b])

    def wait_idx(c, b):
        row = wid * NCH + c
        pltpu.make_async_copy(srcp.at[row, :], srcb[b], semi[b]).wait()
        pltpu.make_async_copy(dstp.at[row, :], dstb[b], semi[b]).wait()
        pltpu.make_async_copy(wsi.at[row, :], wsb[b], semi[b]).wait()

    def issue_gather(b):
        pltpu.async_copy(z.at[srcb[b]], rw[b], semg[b])

    def wait_gather(b):
        pltpu.make_async_copy(z.at[srcb[b]], rw[b], semg[b]).wait()

    def issue_scatter(b):
        pltpu.async_copy(vb[b], h2sh.at[dss[b]], sems[b], add=True)

    def wait_scatter(b):
        pltpu.make_async_copy(vb[b], h2sh.at[dss[b]], sems[b]).wait()

    def compute(b):
        for i in range(CH // 16):
            sl = pl.ds(i * 16, 16)
            dss[b][sl] = dstb[b][sl]

        def edge(p, cc):
            for u in range(4):
                j = p * 4 + u
                jv = jnp.full((16,), j, i32)
                bw = plsc.load_gather(wsb[b], [jv])
                for k in range(H // 16):
                    sl = pl.ds(k * 16, 16)
                    vb[b][j, sl] = rw[b][j, sl] * bw
            return cc

        lax.fori_loop(0, CH // 4, edge, 0)

    issue_idx(0, 0)
    wait_idx(0, 0)
    issue_gather(0)
    issue_idx(1, 1)

    def pair(p, carry):
        for b in (0, 1):
            c = 2 * p + b
            bn = 1 - b
            wait_gather(b)

            @pl.when(c + 1 < NCH)
            def _():
                wait_idx(c + 1, bn)
                issue_gather(bn)

            @pl.when(c >= 2)
            def _():
                wait_scatter(b)

            compute(b)
            issue_scatter(b)

            @pl.when(c + 2 < NCH)
            def _():
                issue_idx(c + 2, b)
        return carry

    lax.fori_loop(0, NCH // 2, pair, 0)
    wait_scatter(0)
    wait_scatter(1)
    plsc.subcore_barrier()

    def readout(k, carry):
        isl = pl.ds(sid * ROWS_PER_SUB + k * CH, CH)
        osl = pl.ds(cid * NP + sid * ROWS_PER_SUB + k * CH, CH)
        pltpu.sync_copy(h2sh.at[isl, :], vb0)
        pltpu.sync_copy(vb0, h2o.at[osl, :])
        return carry

    lax.fori_loop(0, ROWS_PER_SUB // CH, readout, 0)


def _t1_body(x_ref, w_ref, o_ref):
    o_ref[...] = jnp.dot(x_ref[...], w_ref[...], preferred_element_type=f32)


_t1 = pl.pallas_call(
    _t1_body,
    grid=(NP // 1024,),
    in_specs=[pl.BlockSpec((1024, D), lambda i: (i, 0)),
              pl.BlockSpec((D, 8 * H), lambda i: (0, 0))],
    out_specs=pl.BlockSpec((1024, 8 * H), lambda i: (i, 0)),
    out_shape=jax.ShapeDtypeStruct((NP, 8 * H), f32),
)


def _tinv_body(c1_ref, c2_ref, o1_ref, o2_ref):
    o1_ref[...] = 1.0 / jnp.maximum(
        jnp.sum(c1_ref[...], axis=0, keepdims=True), 1.0)
    o2_ref[...] = 1.0 / jnp.maximum(
        jnp.sum(c2_ref[...], axis=0, keepdims=True), 1.0)


_tinv = pl.pallas_call(
    _tinv_body,
    out_shape=(jax.ShapeDtypeStruct((1, CNTN), f32),
               jax.ShapeDtypeStruct((1, CNTN), f32)),
)


def _t2_body(a_ref, b_ref, b1_ref, w2_ref, o_ref):
    h = jnp.maximum(a_ref[...] + b_ref[...] + b1_ref[...], 0.0)
    o_ref[...] = jnp.dot(h, w2_ref[...], preferred_element_type=f32)


_t2 = pl.pallas_call(
    _t2_body,
    grid=(NP // 1024,),
    in_specs=[pl.BlockSpec((1024, H), lambda i: (i, 0)),
              pl.BlockSpec((1024, H), lambda i: (i, 0)),
              pl.BlockSpec((1, H), lambda i: (0, 0)),
              pl.BlockSpec((H, H), lambda i: (0, 0))],
    out_specs=pl.BlockSpec((1024, H), lambda i: (i, 0)),
    out_shape=jax.ShapeDtypeStruct((NP, H), f32),
)


def _t3_body(a_ref, b_ref, b2_ref, wl_ref, bl_ref, o_ref):
    h = a_ref[...] + b_ref[...] + b2_ref[...]
    lg = jnp.dot(h, wl_ref[...], preferred_element_type=f32) + bl_ref[...]
    m = jnp.max(lg, axis=-1, keepdims=True)
    lse = m + jnp.log(jnp.sum(jnp.exp(lg - m), axis=-1, keepdims=True))
    o_ref[...] = lg - lse


_t3 = pl.pallas_call(
    _t3_body,
    grid=(NP // 1024,),
    in_specs=[pl.BlockSpec((1024, H), lambda i: (i, 0)),
              pl.BlockSpec((1024, H), lambda i: (i, 0)),
              pl.BlockSpec((1, H), lambda i: (0, 0)),
              pl.BlockSpec((H, 128), lambda i: (0, 0)),
              pl.BlockSpec((1, 128), lambda i: (0, 0))],
    out_specs=pl.BlockSpec((1024, 128), lambda i: (i, 0)),
    out_shape=jax.ShapeDtypeStruct((NP, 128), f32),
)


def kernel(x, edge_index, emb1, emb2, W1, b1, W2, b2, Wl, bl):
    src = edge_index[0].astype(i32)
    dst = edge_index[1].astype(i32)
    srcp = jnp.pad(src, (0, EP - E), constant_values=N).reshape(EPC, CH)
    dstp = jnp.pad(dst, (0, EP - E), constant_values=N).reshape(EPC, CH)
    e1f = jnp.pad(emb1, ((0, 8), (0, 0))).reshape(-1)
    e2f = jnp.pad(emb2, ((0, 8), (0, 0))).reshape(-1)
    zrows = jnp.zeros((NP, H), f32)

    seg1, seg2, gpk, c1p, c2p = _pass_a(srcp, dstp, e1f, e2f)

    xp = jnp.pad(x, ((0, NP - N), (0, 0)))
    W1b = W1.reshape(8, D, H).transpose(1, 0, 2).reshape(D, 8 * H)
    y = _t1(xp, W1b)
    y8 = y.reshape(YR, H)

    ic1m, ic2m = _tinv(c1p.reshape(NC, CNTN), c2p.reshape(NC, CNTN))
    wpk, ws = _pass_a2(seg1, seg2, ic1m.reshape(CNTN), ic2m.reshape(CNTN))

    h1p = _pass_b(gpk, dstp, wpk, y8, zrows).reshape(NC, NP, H)
    z = _t2(h1p[0], h1p[1], b1.reshape(1, H), W2 * 0.125)

    h2p = _pass_c(srcp, dstp, ws, z, zrows).reshape(NC, NP, H)
    Wlp = jnp.pad(Wl, ((0, 0), (0, 128 - NCLS)))
    blp = jnp.pad(bl, (0, 128 - NCLS), constant_values=-1e30).reshape(1, 128)
    out = _t3(h2p[0], h2p[1], b2.reshape(1, H), Wlp, blp)
    return out[:N, :NCLS]


# speedup vs baseline: 8.0632x; 1.0006x over previous
import functools

import jax
import jax.numpy as jnp
from jax import lax
from jax.experimental import pallas as pl
from jax.experimental.pallas import tpu as pltpu
from jax.experimental.pallas import tpu_sc as plsc

f32 = jnp.float32
i32 = jnp.int32

N = 10000
D = 128
H = 64
NCLS = H // 2
NP = 10240
YR = NP * 8
E = 320000
NC, NS = 2, 16
NW = NC * NS
CH = 128
NCH = 80
EW = CH * NCH
EP = EW * NW
EPC = EP // CH
CNTN = 40960
CNT_PER_SUB = CNTN // NS
ROWS_PER_SUB = NP // NS

_mesh = plsc.VectorSubcoreMesh(core_axis_name="c", subcore_axis_name="s")
_params = pltpu.CompilerParams(needs_layout_passes=False,
                               use_tc_tiling_on_sc=False)


@functools.partial(
    pl.kernel,
    mesh=_mesh,
    compiler_params=_params,
    out_type=(
        jax.ShapeDtypeStruct((EPC, CH), i32),
        jax.ShapeDtypeStruct((EPC, CH), i32),
        jax.ShapeDtypeStruct((EPC, 2 * CH), i32),
        jax.ShapeDtypeStruct((NC * CNTN,), f32),
        jax.ShapeDtypeStruct((NC * CNTN,), f32),
    ),
    scratch_types=(
        pltpu.VMEM((2 * (N + 8),), f32),
        pltpu.VMEM((2 * (N + 8),), f32),
        pltpu.VMEM((CH,), i32),
        pltpu.VMEM((CH,), i32),
        pltpu.VMEM((CH,), i32),
        pltpu.VMEM((CH,), i32),
        pltpu.VMEM((CH,), i32),
        pltpu.VMEM((CH,), i32),
        pltpu.VMEM((CH,), f32),
        pltpu.VMEM((CH,), f32),
        pltpu.VMEM_SHARED((CNTN,), f32),
        pltpu.VMEM_SHARED((CNTN,), f32),
        pltpu.SemaphoreType.DMA,
    ),
)
def _pass_a(srcp, dstp, e1f, e2f,
            seg1o, seg2o, gpo, c1o, c2o,
            e1v, e2v, srcv, dstv, s1v, s2v, g1v, g2v, onesv, zb, c1sh, c2sh,
            sem):
    cid = lax.axis_index("c")
    sid = lax.axis_index("s")
    wid = sid * NC + cid

    pltpu.sync_copy(e1f, e1v)
    pltpu.sync_copy(e2f, e2v)
    for i in range(CH // 16):
        onesv[pl.ds(i * 16, 16)] = jnp.full((16,), 1.0, f32)
        zb[pl.ds(i * 16, 16)] = jnp.zeros((16,), f32)

    def zinit(k, carry):
        zsl = pl.ds(sid * CNT_PER_SUB + k * CH, CH)
        pltpu.sync_copy(zb, c1sh.at[zsl])
        pltpu.sync_copy(zb, c2sh.at[zsl])
        return carry

    lax.fori_loop(0, CNT_PER_SUB // CH, zinit, 0)
    plsc.subcore_barrier()

    def chunk(c, carry):
        row = wid * NCH + c
        pltpu.sync_copy(srcp.at[row, :], srcv)
        pltpu.sync_copy(dstp.at[row, :], dstv)
        two = jnp.full((16,), 2, i32)
        one = jnp.full((16,), 1, i32)
        zero = jnp.zeros((16,), i32)
        for i in range(CH // 16):
            sl = pl.ds(i * 16, 16)
            sv = srcv[sl]
            dv = dstv[sl]
            e1sx = plsc.load_gather(e1v, [sv * 2])
            e1sy = plsc.load_gather(e1v, [sv * 2 + 1])
            e1dx = plsc.load_gather(e1v, [dv * 2])
            e1dy = plsc.load_gather(e1v, [dv * 2 + 1])
            e2sx = plsc.load_gather(e2v, [sv * 2])
            e2sy = plsc.load_gather(e2v, [sv * 2 + 1])
            e2dx = plsc.load_gather(e2v, [dv * 2])
            e2dy = plsc.load_gather(e2v, [dv * 2 + 1])
            r1 = (jnp.where(e1dx - e1sx >= 0.0, two, zero)
                  + jnp.where(e1dy - e1sy >= 0.0, one, zero))
            r2 = (jnp.where(e2dx - e2sx >= 0.0, two, zero)
                  + jnp.where(e2dy - e2sy >= 0.0, one, zero))
            s1v[sl] = r1 * N + dv
            s2v[sl] = r2 * N + dv
            g1v[sl] = sv * 8 + r1
            g2v[sl] = sv * 8 + 4 + r2
        pltpu.sync_copy(s1v, seg1o.at[row, :])
        pltpu.sync_copy(s2v, seg2o.at[row, :])
        pltpu.sync_copy(g1v, gpo.at[row, pl.ds(0, CH)])
        pltpu.sync_copy(g2v, gpo.at[row, pl.ds(CH, CH)])
        pltpu.sync_copy(onesv, c1sh.at[s1v], add=True)
        pltpu.sync_copy(onesv, c2sh.at[s2v], add=True)
        return carry

    lax.fori_loop(0, NCH, chunk, 0)
    plsc.subcore_barrier()

    def readout(k, carry):
        isl = pl.ds(sid * CNT_PER_SUB + k * CH, CH)
        osl = pl.ds(cid * CNTN + sid * CNT_PER_SUB + k * CH, CH)
        pltpu.sync_copy(c1sh.at[isl], zb)
        pltpu.sync_copy(zb, c1o.at[osl])
        pltpu.sync_copy(c2sh.at[isl], zb)
        pltpu.sync_copy(zb, c2o.at[osl])
        return carry

    lax.fori_loop(0, CNT_PER_SUB // CH, readout, 0)


@functools.partial(
    pl.kernel,
    mesh=_mesh,
    compiler_params=_params,
    out_type=(
        jax.ShapeDtypeStruct((EPC, 2 * CH), f32),
        jax.ShapeDtypeStruct((EPC, CH), f32),
    ),
    scratch_types=(
        pltpu.VMEM((CNTN,), f32),
        pltpu.VMEM((CNTN,), f32),
        pltpu.VMEM((CH,), i32),
        pltpu.VMEM((CH,), i32),
        pltpu.VMEM((CH,), f32),
        pltpu.VMEM((CH,), f32),
        pltpu.VMEM((CH,), f32),
        pltpu.SemaphoreType.DMA,
    ),
)
def _pass_a2(s1i, s2i, ic1, ic2,
             wpo, wso,
             ic1v, ic2v, s1v, s2v, w1v, w2v, wsv, sem):
    cid = lax.axis_index("c")
    sid = lax.axis_index("s")
    wid = sid * NC + cid

    pltpu.sync_copy(ic1, ic1v)
    pltpu.sync_copy(ic2, ic2v)

    def chunk(c, carry):
        row = wid * NCH + c
        pltpu.sync_copy(s1i.at[row, :], s1v)
        pltpu.sync_copy(s2i.at[row, :], s2v)
        for i in range(CH // 16):
            sl = pl.ds(i * 16, 16)
            w1 = plsc.load_gather(ic1v, [s1v[sl]])
            w2 = plsc.load_gather(ic2v, [s2v[sl]])
            w1v[sl] = w1
            w2v[sl] = w2
            wsv[sl] = w1 + w2
        pltpu.sync_copy(w1v, wpo.at[row, pl.ds(0, CH)])
        pltpu.sync_copy(w2v, wpo.at[row, pl.ds(CH, CH)])
        pltpu.sync_copy(wsv, wso.at[row, :])
        return carry

    lax.fori_loop(0, NCH, chunk, 0)


@functools.partial(
    pl.kernel,
    mesh=_mesh,
    compiler_params=_params,
    out_type=jax.ShapeDtypeStruct((NC * NP, H), f32),
    scratch_types=(
        pltpu.VMEM((2 * CH,), i32),
        pltpu.VMEM((2 * CH,), i32),
        pltpu.VMEM((CH,), i32),
        pltpu.VMEM((CH,), i32),
        pltpu.VMEM((2 * CH,), f32),
        pltpu.VMEM((2 * CH,), f32),
        pltpu.VMEM((CH,), i32),
        pltpu.VMEM((CH,), i32),
        pltpu.VMEM((2 * CH, H), f32),
        pltpu.VMEM((2 * CH, H), f32),
        pltpu.VMEM((CH, H), f32),
        pltpu.VMEM((CH, H), f32),
        pltpu.VMEM_SHARED((NP, H), f32),
        pltpu.SemaphoreType.DMA,
        pltpu.SemaphoreType.DMA,
        pltpu.SemaphoreType.DMA,
        pltpu.SemaphoreType.DMA,
        pltpu.SemaphoreType.DMA,
        pltpu.SemaphoreType.DMA,
    ),
)
def _pass_b(gpi, dsti, wpi, y8, zrows,
            h1o,
            gp0, gp1, dst0, dst1, wp0, wp1, ds0, ds1, rw0, rw1, vb0, vb1,
            h1sh, semi0, semi1, semg0, semg1, sems0, sems1):
    cid = lax.axis_index("c")
    sid = lax.axis_index("s")
    wid = sid * NC + cid

    gp = (gp0, gp1)
    dstb = (dst0, dst1)
    wp = (wp0, wp1)
    dss = (ds0, ds1)
    rw = (rw0, rw1)
    vb = (vb0, vb1)
    semi = (semi0, semi1)
    semg = (semg0, semg1)
    sems = (sems0, sems1)

    def zinit(k, carry):
        rsl = pl.ds(sid * ROWS_PER_SUB + k * CH, CH)
        pltpu.sync_copy(zrows.at[rsl, :], vb0)
        pltpu.sync_copy(vb0, h1sh.at[rsl, :])
        return carry

    lax.fori_loop(0, ROWS_PER_SUB // CH, zinit, 0)
    plsc.subcore_barrier()

    def issue_idx(c, b):
        row = wid * NCH + c
        pltpu.async_copy(gpi.at[row, :], gp[b], semi[b])
        pltpu.async_copy(dsti.at[row, :], dstb[b], semi[b])
        pltpu.async_copy(wpi.at[row, :], wp[b], semi[b])

    def wait_idx(c, b):
        row = wid * NCH + c
        pltpu.make_async_copy(gpi.at[row, :], gp[b], semi[b]).wait()
        pltpu.make_async_copy(dsti.at[row, :], dstb[b], semi[b]).wait()
        pltpu.make_async_copy(wpi.at[row, :], wp[b], semi[b]).wait()

    def issue_gather(b):
        pltpu.async_copy(y8.at[gp[b]], rw[b], semg[b])

    def wait_gather(b):
        pltpu.make_async_copy(y8.at[gp[b]], rw[b], semg[b]).wait()

    def issue_scatter(b):
        pltpu.async_copy(vb[b], h1sh.at[dss[b]], sems[b], add=True)

    def wait_scatter(b):
        pltpu.make_async_copy(vb[b], h1sh.at[dss[b]], sems[b]).wait()

    def compute(b):
        for i in range(CH // 16):
            sl = pl.ds(i * 16, 16)
            dss[b][sl] = dstb[b][sl]

        def edge(p, cc):
            for u in range(8):
                j = p * 8 + u
                jv = jnp.full((16,), j, i32)
                bw1 = plsc.load_gather(wp[b], [jv])
                bw2 = plsc.load_gather(wp[b], [jv + CH])
                for k in range(H // 16):
                    sl = pl.ds(k * 16, 16)
                    vb[b][j, sl] = (rw[b][j, sl] * bw1
                                    + rw[b][j + CH, sl] * bw2)
            return cc

        lax.fori_loop(0, CH // 8, edge, 0)

    issue_idx(0, 0)
    wait_idx(0, 0)
    issue_gather(0)
    issue_idx(1, 1)

    def pair(p, carry):
        for b in (0, 1):
            c = 2 * p + b
            bn = 1 - b
            wait_gather(b)

            @pl.when(c + 1 < NCH)
            def _():
                wait_idx(c + 1, bn)
                issue_gather(bn)

            @pl.when(c >= 2)
            def _():
                wait_scatter(b)

            compute(b)
            issue_scatter(b)

            @pl.when(c + 2 < NCH)
            def _():
                issue_idx(c + 2, b)
        return carry

    lax.fori_loop(0, NCH // 2, pair, 0)
    wait_scatter(0)
    wait_scatter(1)
    plsc.subcore_barrier()

    def readout(k, carry):
        isl = pl.ds(sid * ROWS_PER_SUB + k * CH, CH)
        osl = pl.ds(cid * NP + sid * ROWS_PER_SUB + k * CH, CH)
        pltpu.sync_copy(h1sh.at[isl, :], vb0)
        pltpu.sync_copy(vb0, h1o.at[osl, :])
        return carry

    lax.fori_loop(0, ROWS_PER_SUB // CH, readout, 0)


@functools.partial(
    pl.kernel,
    mesh=_mesh,
    compiler_params=_params,
    out_type=jax.ShapeDtypeStruct((NC * NP, H), f32),
    scratch_types=(
        pltpu.VMEM((CH,), i32),
        pltpu.VMEM((CH,), i32),
        pltpu.VMEM((CH,), i32),
        pltpu.VMEM((CH,), i32),
        pltpu.VMEM((CH,), f32),
        pltpu.VMEM((CH,), f32),
        pltpu.VMEM((CH,), i32),
        pltpu.VMEM((CH,), i32),
        pltpu.VMEM((CH, H), f32),
        pltpu.VMEM((CH, H), f32),
        pltpu.VMEM((CH, H), f32),
        pltpu.VMEM((CH, H), f32),
        pltpu.VMEM_SHARED((NP, H), f32),
        pltpu.SemaphoreType.DMA,
        pltpu.SemaphoreType.DMA,
        pltpu.SemaphoreType.DMA,
        pltpu.SemaphoreType.DMA,
        pltpu.SemaphoreType.DMA,
        pltpu.SemaphoreType.DMA,
    ),
)
def _pass_c(srcp, dstp, wsi, z, zrows,
            h2o,
            src0, src1, dst0, dst1, ws0, ws1, ds0, ds1, rw0, rw1, vb0, vb1,
            h2sh, semi0, semi1, semg0, semg1, sems0, sems1):
    cid = lax.axis_index("c")
    sid = lax.axis_index("s")
    wid = sid * NC + cid

    srcb = (src0, src1)
    dstb = (dst0, dst1)
    wsb = (ws0, ws1)
    dss = (ds0, ds1)
    rw = (rw0, rw1)
    vb = (vb0, vb1)
    semi = (semi0, semi1)
    semg = (semg0, semg1)
    sems = (sems0, sems1)

    def zinit(k, carry):
        rsl = pl.ds(sid * ROWS_PER_SUB + k * CH, CH)
        pltpu.sync_copy(zrows.at[rsl, :], vb0)
        pltpu.sync_copy(vb0, h2sh.at[rsl, :])
        return carry

    lax.fori_loop(0, ROWS_PER_SUB // CH, zinit, 0)
    plsc.subcore_barrier()

    def issue_idx(c, b):
        row = wid * NCH + c
        pltpu.async_copy(srcp.at[row, :], srcb[b], semi[b])
        pltpu.async_copy(dstp.at[row, :], dstb[b], semi[b])
        pltpu.async_copy(wsi.at[row, :], wsb[b], semi[b])

    def wait_idx(c, b):
        row = wid * NCH + c
        pltpu.make_async_copy(srcp.at[row, :], srcb[b], semi[b]).wait()
        pltpu.make_async_copy(dstp.at[row, :], dstb[b], semi[b]).wait()
        pltpu.make_async_copy(wsi.at[row, :], wsb[b], semi[b]).wait()

    def issue_gather(b):
        pltpu.async_copy(z.at[srcb[b]], rw[b], semg[b])

    def wait_gather(b):
        pltpu.make_async_copy(z.at[srcb[b]], rw[b], semg[b]).wait()

    def issue_scatter(b):
        pltpu.async_copy(vb[b], h2sh.at[dss[b]], sems[b], add=True)

    def wait_scatter(b):
        pltpu.make_async_copy(vb[b], h2sh.at[dss[b]], sems[b]).wait()

    def compute(b):
        for i in range(CH // 16):
            sl = pl.ds(i * 16, 16)
            dss[b][sl] = dstb[b][sl]

        def edge(p, cc):
            for u in range(8):
                j = p * 8 + u
                jv = jnp.full((16,), j, i32)
                bw = plsc.load_gather(wsb[b], [jv])
                for k in range(H // 16):
                    sl = pl.ds(k * 16, 16)
                    vb[b][j, sl] = rw[b][j, sl] * bw
            return cc

        lax.fori_loop(0, CH // 8, edge, 0)

    issue_idx(0, 0)
    wait_idx(0, 0)
    issue_gather(0)
    issue_idx(1, 1)

    def pair(p, carry):
        for b in (0, 1):
            c = 2 * p + b
            bn = 1 - b
            wait_gather(b)

            @pl.when(c + 1 < NCH)
            def _():
                wait_idx(c + 1, bn)
                issue_gather(bn)

            @pl.when(c >= 2)
            def _():
                wait_scatter(b)

            compute(b)
            issue_scatter(b)

            @pl.when(c + 2 < NCH)
            def _():
                issue_idx(c + 2, b)
        return carry

    lax.fori_loop(0, NCH // 2, pair, 0)
    wait_scatter(0)
    wait_scatter(1)
    plsc.subcore_barrier()

    def readout(k, carry):
        isl = pl.ds(sid * ROWS_PER_SUB + k * CH, CH)
        osl = pl.ds(cid * NP + sid * ROWS_PER_SUB + k * CH, CH)
        pltpu.sync_copy(h2sh.at[isl, :], vb0)
        pltpu.sync_copy(vb0, h2o.at[osl, :])
        return carry

    lax.fori_loop(0, ROWS_PER_SUB // CH, readout, 0)


def _t1_body(x_ref, w_ref, o_ref):
    o_ref[...] = jnp.dot(x_ref[...], w_ref[...], preferred_element_type=f32)


_t1 = pl.pallas_call(
    _t1_body,
    grid=(NP // 1024,),
    in_specs=[pl.BlockSpec((1024, D), lambda i: (i, 0)),
              pl.BlockSpec((D, 8 * H), lambda i: (0, 0))],
    out_specs=pl.BlockSpec((1024, 8 * H), lambda i: (i, 0)),
    out_shape=jax.ShapeDtypeStruct((NP, 8 * H), f32),
)


def _tinv_body(c1_ref, c2_ref, o1_ref, o2_ref):
    o1_ref[...] = 1.0 / jnp.maximum(
        jnp.sum(c1_ref[...], axis=0, keepdims=True), 1.0)
    o2_ref[...] = 1.0 / jnp.maximum(
        jnp.sum(c2_ref[...], axis=0, keepdims=True), 1.0)


_tinv = pl.pallas_call(
    _tinv_body,
    out_shape=(jax.ShapeDtypeStruct((1, CNTN), f32),
               jax.ShapeDtypeStruct((1, CNTN), f32)),
)


def _t2_body(a_ref, b_ref, b1_ref, w2_ref, o_ref):
    h = jnp.maximum(a_ref[...] + b_ref[...] + b1_ref[...], 0.0)
    o_ref[...] = jnp.dot(h, w2_ref[...], preferred_element_type=f32)


_t2 = pl.pallas_call(
    _t2_body,
    grid=(NP // 1024,),
    in_specs=[pl.BlockSpec((1024, H), lambda i: (i, 0)),
              pl.BlockSpec((1024, H), lambda i: (i, 0)),
              pl.BlockSpec((1, H), lambda i: (0, 0)),
              pl.BlockSpec((H, H), lambda i: (0, 0))],
    out_specs=pl.BlockSpec((1024, H), lambda i: (i, 0)),
    out_shape=jax.ShapeDtypeStruct((NP, H), f32),
)


def _t3_body(a_ref, b_ref, b2_ref, wl_ref, bl_ref, o_ref):
    h = a_ref[...] + b_ref[...] + b2_ref[...]
    lg = jnp.dot(h, wl_ref[...], preferred_element_type=f32) + bl_ref[...]
    m = jnp.max(lg, axis=-1, keepdims=True)
    lse = m + jnp.log(jnp.sum(jnp.exp(lg - m), axis=-1, keepdims=True))
    o_ref[...] = lg - lse


_t3 = pl.pallas_call(
    _t3_body,
    grid=(NP // 1024,),
    in_specs=[pl.BlockSpec((1024, H), lambda i: (i, 0)),
              pl.BlockSpec((1024, H), lambda i: (i, 0)),
              pl.BlockSpec((1, H), lambda i: (0, 0)),
              pl.BlockSpec((H, 128), lambda i: (0, 0)),
              pl.BlockSpec((1, 128), lambda i: (0, 0))],
    out_specs=pl.BlockSpec((1024, 128), lambda i: (i, 0)),
    out_shape=jax.ShapeDtypeStruct((NP, 128), f32),
)


def kernel(x, edge_index, emb1, emb2, W1, b1, W2, b2, Wl, bl):
    src = edge_index[0].astype(i32)
    dst = edge_index[1].astype(i32)
    srcp = jnp.pad(src, (0, EP - E), constant_values=N).reshape(EPC, CH)
    dstp = jnp.pad(dst, (0, EP - E), constant_values=N).reshape(EPC, CH)
    e1f = jnp.pad(emb1, ((0, 8), (0, 0))).reshape(-1)
    e2f = jnp.pad(emb2, ((0, 8), (0, 0))).reshape(-1)
    zrows = jnp.zeros((NP, H), f32)

    seg1, seg2, gpk, c1p, c2p = _pass_a(srcp, dstp, e1f, e2f)

    xp = jnp.pad(x, ((0, NP - N), (0, 0)))
    W1b = W1.reshape(8, D, H).transpose(1, 0, 2).reshape(D, 8 * H)
    y = _t1(xp, W1b)
    y8 = y.reshape(YR, H)

    ic1m, ic2m = _tinv(c1p.reshape(NC, CNTN), c2p.reshape(NC, CNTN))
    wpk, ws = _pass_a2(seg1, seg2, ic1m.reshape(CNTN), ic2m.reshape(CNTN))

    h1p = _pass_b(gpk, dstp, wpk, y8, zrows).reshape(NC, NP, H)
    z = _t2(h1p[0], h1p[1], b1.reshape(1, H), W2 * 0.125)

    h2p = _pass_c(srcp, dstp, ws, z, zrows).reshape(NC, NP, H)
    Wlp = jnp.pad(Wl, ((0, 0), (0, 128 - NCLS)))
    blp = jnp.pad(bl, (0, 128 - NCLS), constant_values=-1e30).reshape(1, 128)
    out = _t3(h2p[0], h2p[1], b2.reshape(1, H), Wlp, blp)
    return out[:N, :NCLS]


# speedup vs baseline: 8.8246x; 1.0944x over previous
import functools

import jax
import jax.numpy as jnp
from jax import lax
from jax.experimental import pallas as pl
from jax.experimental.pallas import tpu as pltpu
from jax.experimental.pallas import tpu_sc as plsc

f32 = jnp.float32
i32 = jnp.int32

N = 10000
D = 128
H = 64
NCLS = H // 2
NP = 10240
YR = NP * 8
E = 320000
NC, NS = 2, 16
NW = NC * NS
CH = 128
NCH = 80
EW = CH * NCH
EP = EW * NW
EPC = EP // CH
CNTN = 40960
CNT_PER_SUB = CNTN // NS
ROWS_PER_SUB = NP // NS

_mesh = plsc.VectorSubcoreMesh(core_axis_name="c", subcore_axis_name="s")
_params = pltpu.CompilerParams(needs_layout_passes=False,
                               use_tc_tiling_on_sc=False)


@functools.partial(
    pl.kernel,
    mesh=_mesh,
    compiler_params=_params,
    out_type=(
        jax.ShapeDtypeStruct((EPC, CH), i32),
        jax.ShapeDtypeStruct((EPC, CH), i32),
        jax.ShapeDtypeStruct((EPC, 2 * CH), i32),
        jax.ShapeDtypeStruct((NC * CNTN,), f32),
        jax.ShapeDtypeStruct((NC * CNTN,), f32),
    ),
    scratch_types=(
        pltpu.VMEM((2 * (N + 8),), f32),
        pltpu.VMEM((2 * (N + 8),), f32),
        pltpu.VMEM((CH,), i32),
        pltpu.VMEM((CH,), i32),
        pltpu.VMEM((CH,), i32),
        pltpu.VMEM((CH,), i32),
        pltpu.VMEM((CH,), i32),
        pltpu.VMEM((CH,), i32),
        pltpu.VMEM((CH,), f32),
        pltpu.VMEM((CH,), f32),
        pltpu.VMEM_SHARED((CNTN,), f32),
        pltpu.VMEM_SHARED((CNTN,), f32),
        pltpu.SemaphoreType.DMA,
    ),
)
def _pass_a(srcp, dstp, e1f, e2f,
            seg1o, seg2o, gpo, c1o, c2o,
            e1v, e2v, srcv, dstv, s1v, s2v, g1v, g2v, onesv, zb, c1sh, c2sh,
            sem):
    cid = lax.axis_index("c")
    sid = lax.axis_index("s")
    wid = sid * NC + cid

    pltpu.sync_copy(e1f, e1v)
    pltpu.sync_copy(e2f, e2v)
    for i in range(CH // 16):
        onesv[pl.ds(i * 16, 16)] = jnp.full((16,), 1.0, f32)
        zb[pl.ds(i * 16, 16)] = jnp.zeros((16,), f32)

    def zinit(k, carry):
        zsl = pl.ds(sid * CNT_PER_SUB + k * CH, CH)
        pltpu.sync_copy(zb, c1sh.at[zsl])
        pltpu.sync_copy(zb, c2sh.at[zsl])
        return carry

    lax.fori_loop(0, CNT_PER_SUB // CH, zinit, 0)
    plsc.subcore_barrier()

    def chunk(c, carry):
        row = wid * NCH + c
        pltpu.sync_copy(srcp.at[row, :], srcv)
        pltpu.sync_copy(dstp.at[row, :], dstv)
        two = jnp.full((16,), 2, i32)
        one = jnp.full((16,), 1, i32)
        zero = jnp.zeros((16,), i32)
        for i in range(CH // 16):
            sl = pl.ds(i * 16, 16)
            sv = srcv[sl]
            dv = dstv[sl]
            e1sx = plsc.load_gather(e1v, [sv * 2])
            e1sy = plsc.load_gather(e1v, [sv * 2 + 1])
            e1dx = plsc.load_gather(e1v, [dv * 2])
            e1dy = plsc.load_gather(e1v, [dv * 2 + 1])
            e2sx = plsc.load_gather(e2v, [sv * 2])
            e2sy = plsc.load_gather(e2v, [sv * 2 + 1])
            e2dx = plsc.load_gather(e2v, [dv * 2])
            e2dy = plsc.load_gather(e2v, [dv * 2 + 1])
            r1 = (jnp.where(e1dx - e1sx >= 0.0, two, zero)
                  + jnp.where(e1dy - e1sy >= 0.0, one, zero))
            r2 = (jnp.where(e2dx - e2sx >= 0.0, two, zero)
                  + jnp.where(e2dy - e2sy >= 0.0, one, zero))
            s1v[sl] = r1 * N + dv
            s2v[sl] = r2 * N + dv
            g1v[sl] = sv * 8 + r1
            g2v[sl] = sv * 8 + 4 + r2
        pltpu.sync_copy(s1v, seg1o.at[row, :])
        pltpu.sync_copy(s2v, seg2o.at[row, :])
        pltpu.sync_copy(g1v, gpo.at[row, pl.ds(0, CH)])
        pltpu.sync_copy(g2v, gpo.at[row, pl.ds(CH, CH)])
        pltpu.sync_copy(onesv, c1sh.at[s1v], add=True)
        pltpu.sync_copy(onesv, c2sh.at[s2v], add=True)
        return carry

    lax.fori_loop(0, NCH, chunk, 0)
    plsc.subcore_barrier()

    def readout(k, carry):
        isl = pl.ds(sid * CNT_PER_SUB + k * CH, CH)
        osl = pl.ds(cid * CNTN + sid * CNT_PER_SUB + k * CH, CH)
        pltpu.sync_copy(c1sh.at[isl], zb)
        pltpu.sync_copy(zb, c1o.at[osl])
        pltpu.sync_copy(c2sh.at[isl], zb)
        pltpu.sync_copy(zb, c2o.at[osl])
        return carry

    lax.fori_loop(0, CNT_PER_SUB // CH, readout, 0)


@functools.partial(
    pl.kernel,
    mesh=_mesh,
    compiler_params=_params,
    out_type=jax.ShapeDtypeStruct((NC * NP, H), f32),
    scratch_types=(
        pltpu.VMEM((2 * CH,), i32),
        pltpu.VMEM((2 * CH,), i32),
        pltpu.VMEM((CH,), i32),
        pltpu.VMEM((CH,), i32),
        pltpu.VMEM((CH,), i32),
        pltpu.VMEM((CH,), i32),
        pltpu.VMEM((CH,), i32),
        pltpu.VMEM((CH,), i32),
        pltpu.VMEM((CH,), f32),
        pltpu.VMEM((CH,), f32),
        pltpu.VMEM((CH,), f32),
        pltpu.VMEM((CH,), f32),
        pltpu.VMEM((CH,), i32),
        pltpu.VMEM((CH,), i32),
        pltpu.VMEM((2 * CH, H), f32),
        pltpu.VMEM((2 * CH, H), f32),
        pltpu.VMEM((CH, H), f32),
        pltpu.VMEM((CH, H), f32),
        pltpu.VMEM_SHARED((NP, H), f32),
        pltpu.SemaphoreType.DMA,
        pltpu.SemaphoreType.DMA,
        pltpu.SemaphoreType.DMA,
        pltpu.SemaphoreType.DMA,
        pltpu.SemaphoreType.DMA,
        pltpu.SemaphoreType.DMA,
    ),
)
def _pass_b(gpi, dsti, s1i, s2i, ic1, ic2, y8, zrows,
            h1o,
            gp0, gp1, dst0, dst1, s10, s11, s20, s21, w10, w11, w20, w21,
            ds0, ds1, rw0, rw1, vb0, vb1,
            h1sh, semi0, semi1, semg0, semg1, sems0, sems1):
    cid = lax.axis_index("c")
    sid = lax.axis_index("s")
    wid = sid * NC + cid

    gp = (gp0, gp1)
    dstb = (dst0, dst1)
    s1b = (s10, s11)
    s2b = (s20, s21)
    w1b = (w10, w11)
    w2b = (w20, w21)
    dss = (ds0, ds1)
    rw = (rw0, rw1)
    vb = (vb0, vb1)
    semi = (semi0, semi1)
    semg = (semg0, semg1)
    sems = (sems0, sems1)

    def zinit(k, carry):
        rsl = pl.ds(sid * ROWS_PER_SUB + k * CH, CH)
        pltpu.sync_copy(zrows.at[rsl, :], vb0)
        pltpu.sync_copy(vb0, h1sh.at[rsl, :])
        return carry

    lax.fori_loop(0, ROWS_PER_SUB // CH, zinit, 0)
    plsc.subcore_barrier()

    def issue_idx(c, b):
        row = wid * NCH + c
        pltpu.async_copy(gpi.at[row, :], gp[b], semi[b])
        pltpu.async_copy(dsti.at[row, :], dstb[b], semi[b])
        pltpu.async_copy(s1i.at[row, :], s1b[b], semi[b])
        pltpu.async_copy(s2i.at[row, :], s2b[b], semi[b])

    def wait_idx(c, b):
        row = wid * NCH + c
        pltpu.make_async_copy(gpi.at[row, :], gp[b], semi[b]).wait()
        pltpu.make_async_copy(dsti.at[row, :], dstb[b], semi[b]).wait()
        pltpu.make_async_copy(s1i.at[row, :], s1b[b], semi[b]).wait()
        pltpu.make_async_copy(s2i.at[row, :], s2b[b], semi[b]).wait()

    def issue_gather(b):
        pltpu.async_copy(y8.at[gp[b]], rw[b], semg[b])
        pltpu.async_copy(ic1.at[s1b[b]], w1b[b], semg[b])
        pltpu.async_copy(ic2.at[s2b[b]], w2b[b], semg[b])

    def wait_gather(b):
        pltpu.make_async_copy(y8.at[gp[b]], rw[b], semg[b]).wait()
        pltpu.make_async_copy(ic1.at[s1b[b]], w1b[b], semg[b]).wait()
        pltpu.make_async_copy(ic2.at[s2b[b]], w2b[b], semg[b]).wait()

    def issue_scatter(b):
        pltpu.async_copy(vb[b], h1sh.at[dss[b]], sems[b], add=True)

    def wait_scatter(b):
        pltpu.make_async_copy(vb[b], h1sh.at[dss[b]], sems[b]).wait()

    def compute(b):
        for i in range(CH // 16):
            sl = pl.ds(i * 16, 16)
            dss[b][sl] = dstb[b][sl]

        def edge(p, cc):
            for u in range(8):
                j = p * 8 + u
                jv = jnp.full((16,), j, i32)
                bw1 = plsc.load_gather(w1b[b], [jv])
                bw2 = plsc.load_gather(w2b[b], [jv])
                for k in range(H // 16):
                    sl = pl.ds(k * 16, 16)
                    vb[b][j, sl] = (rw[b][j, sl] * bw1
                                    + rw[b][j + CH, sl] * bw2)
            return cc

        lax.fori_loop(0, CH // 8, edge, 0)

    issue_idx(0, 0)
    wait_idx(0, 0)
    issue_gather(0)
    issue_idx(1, 1)

    def pair(p, carry):
        for b in (0, 1):
            c = 2 * p + b
            bn = 1 - b
            wait_gather(b)

            @pl.when(c + 1 < NCH)
            def _():
                wait_idx(c + 1, bn)
                issue_gather(bn)

            @pl.when(c >= 2)
            def _():
                wait_scatter(b)

            compute(b)
            issue_scatter(b)

            @pl.when(c + 2 < NCH)
            def _():
                issue_idx(c + 2, b)
        return carry

    lax.fori_loop(0, NCH // 2, pair, 0)
    wait_scatter(0)
    wait_scatter(1)
    plsc.subcore_barrier()

    def readout(k, carry):
        isl = pl.ds(sid * ROWS_PER_SUB + k * CH, CH)
        osl = pl.ds(cid * NP + sid * ROWS_PER_SUB + k * CH, CH)
        pltpu.sync_copy(h1sh.at[isl, :], vb0)
        pltpu.sync_copy(vb0, h1o.at[osl, :])
        return carry

    lax.fori_loop(0, ROWS_PER_SUB // CH, readout, 0)


@functools.partial(
    pl.kernel,
    mesh=_mesh,
    compiler_params=_params,
    out_type=jax.ShapeDtypeStruct((NC * NP, H), f32),
    scratch_types=(
        pltpu.VMEM((CH,), i32),
        pltpu.VMEM((CH,), i32),
        pltpu.VMEM((CH,), i32),
        pltpu.VMEM((CH,), i32),
        pltpu.VMEM((CH,), i32),
        pltpu.VMEM((CH,), i32),
        pltpu.VMEM((CH,), i32),
        pltpu.VMEM((CH,), i32),
        pltpu.VMEM((CH,), f32),
        pltpu.VMEM((CH,), f32),
        pltpu.VMEM((CH,), f32),
        pltpu.VMEM((CH,), f32),
        pltpu.VMEM((CH,), i32),
        pltpu.VMEM((CH,), i32),
        pltpu.VMEM((CH, H), f32),
        pltpu.VMEM((CH, H), f32),
        pltpu.VMEM((CH, H), f32),
        pltpu.VMEM((CH, H), f32),
        pltpu.VMEM_SHARED((NP, H), f32),
        pltpu.SemaphoreType.DMA,
        pltpu.SemaphoreType.DMA,
        pltpu.SemaphoreType.DMA,
        pltpu.SemaphoreType.DMA,
        pltpu.SemaphoreType.DMA,
        pltpu.SemaphoreType.DMA,
    ),
)
def _pass_c(srcp, dstp, s1i, s2i, ic1, ic2, z, zrows,
            h2o,
            src0, src1, dst0, dst1, s10, s11, s20, s21, w10, w11, w20, w21,
            ds0, ds1, rw0, rw1, vb0, vb1,
            h2sh, semi0, semi1, semg0, semg1, sems0, sems1):
    cid = lax.axis_index("c")
    sid = lax.axis_index("s")
    wid = sid * NC + cid

    srcb = (src0, src1)
    dstb = (dst0, dst1)
    s1b = (s10, s11)
    s2b = (s20, s21)
    w1b = (w10, w11)
    w2b = (w20, w21)
    dss = (ds0, ds1)
    rw = (rw0, rw1)
    vb = (vb0, vb1)
    semi = (semi0, semi1)
    semg = (semg0, semg1)
    sems = (sems0, sems1)

    def zinit(k, carry):
        rsl = pl.ds(sid * ROWS_PER_SUB + k * CH, CH)
        pltpu.sync_copy(zrows.at[rsl, :], vb0)
        pltpu.sync_copy(vb0, h2sh.at[rsl, :])
        return carry

    lax.fori_loop(0, ROWS_PER_SUB // CH, zinit, 0)
    plsc.subcore_barrier()

    def issue_idx(c, b):
        row = wid * NCH + c
        pltpu.async_copy(srcp.at[row, :], srcb[b], semi[b])
        pltpu.async_copy(dstp.at[row, :], dstb[b], semi[b])
        pltpu.async_copy(s1i.at[row, :], s1b[b], semi[b])
        pltpu.async_copy(s2i.at[row, :], s2b[b], semi[b])

    def wait_idx(c, b):
        row = wid * NCH + c
        pltpu.make_async_copy(srcp.at[row, :], srcb[b], semi[b]).wait()
        pltpu.make_async_copy(dstp.at[row, :], dstb[b], semi[b]).wait()
        pltpu.make_async_copy(s1i.at[row, :], s1b[b], semi[b]).wait()
        pltpu.make_async_copy(s2i.at[row, :], s2b[b], semi[b]).wait()

    def issue_gather(b):
        pltpu.async_copy(z.at[srcb[b]], rw[b], semg[b])
        pltpu.async_copy(ic1.at[s1b[b]], w1b[b], semg[b])
        pltpu.async_copy(ic2.at[s2b[b]], w2b[b], semg[b])

    def wait_gather(b):
        pltpu.make_async_copy(z.at[srcb[b]], rw[b], semg[b]).wait()
        pltpu.make_async_copy(ic1.at[s1b[b]], w1b[b], semg[b]).wait()
        pltpu.make_async_copy(ic2.at[s2b[b]], w2b[b], semg[b]).wait()

    def issue_scatter(b):
        pltpu.async_copy(vb[b], h2sh.at[dss[b]], sems[b], add=True)

    def wait_scatter(b):
        pltpu.make_async_copy(vb[b], h2sh.at[dss[b]], sems[b]).wait()

    def compute(b):
        for i in range(CH // 16):
            sl = pl.ds(i * 16, 16)
            dss[b][sl] = dstb[b][sl]

        def edge(p, cc):
            for u in range(8):
                j = p * 8 + u
                jv = jnp.full((16,), j, i32)
                bw = (plsc.load_gather(w1b[b], [jv])
                      + plsc.load_gather(w2b[b], [jv]))
                for k in range(H // 16):
                    sl = pl.ds(k * 16, 16)
                    vb[b][j, sl] = rw[b][j, sl] * bw
            return cc

        lax.fori_loop(0, CH // 8, edge, 0)

    issue_idx(0, 0)
    wait_idx(0, 0)
    issue_gather(0)
    issue_idx(1, 1)

    def pair(p, carry):
        for b in (0, 1):
            c = 2 * p + b
            bn = 1 - b
            wait_gather(b)

            @pl.when(c + 1 < NCH)
            def _():
                wait_idx(c + 1, bn)
                issue_gather(bn)

            @pl.when(c >= 2)
            def _():
                wait_scatter(b)

            compute(b)
            issue_scatter(b)

            @pl.when(c + 2 < NCH)
            def _():
                issue_idx(c + 2, b)
        return carry

    lax.fori_loop(0, NCH // 2, pair, 0)
    wait_scatter(0)
    wait_scatter(1)
    plsc.subcore_barrier()

    def readout(k, carry):
        isl = pl.ds(sid * ROWS_PER_SUB + k * CH, CH)
        osl = pl.ds(cid * NP + sid * ROWS_PER_SUB + k * CH, CH)
        pltpu.sync_copy(h2sh.at[isl, :], vb0)
        pltpu.sync_copy(vb0, h2o.at[osl, :])
        return carry

    lax.fori_loop(0, ROWS_PER_SUB // CH, readout, 0)


def _t1_body(x_ref, w_ref, o_ref):
    o_ref[...] = jnp.dot(x_ref[...], w_ref[...], preferred_element_type=f32)


_t1 = pl.pallas_call(
    _t1_body,
    grid=(NP // 1024,),
    in_specs=[pl.BlockSpec((1024, D), lambda i: (i, 0)),
              pl.BlockSpec((D, 8 * H), lambda i: (0, 0))],
    out_specs=pl.BlockSpec((1024, 8 * H), lambda i: (i, 0)),
    out_shape=jax.ShapeDtypeStruct((NP, 8 * H), f32),
)


def _tinv_body(c1_ref, c2_ref, o1_ref, o2_ref):
    o1_ref[...] = 1.0 / jnp.maximum(
        jnp.sum(c1_ref[...], axis=0, keepdims=True), 1.0)
    o2_ref[...] = 1.0 / jnp.maximum(
        jnp.sum(c2_ref[...], axis=0, keepdims=True), 1.0)


_tinv = pl.pallas_call(
    _tinv_body,
    out_shape=(jax.ShapeDtypeStruct((1, CNTN), f32),
               jax.ShapeDtypeStruct((1, CNTN), f32)),
)


def _t2_body(a_ref, b_ref, b1_ref, w2_ref, o_ref):
    h = jnp.maximum(a_ref[...] + b_ref[...] + b1_ref[...], 0.0)
    o_ref[...] = jnp.dot(h, w2_ref[...], preferred_element_type=f32)


_t2 = pl.pallas_call(
    _t2_body,
    grid=(NP // 1024,),
    in_specs=[pl.BlockSpec((1024, H), lambda i: (i, 0)),
              pl.BlockSpec((1024, H), lambda i: (i, 0)),
              pl.BlockSpec((1, H), lambda i: (0, 0)),
              pl.BlockSpec((H, H), lambda i: (0, 0))],
    out_specs=pl.BlockSpec((1024, H), lambda i: (i, 0)),
    out_shape=jax.ShapeDtypeStruct((NP, H), f32),
)


def _t3_body(a_ref, b_ref, b2_ref, wl_ref, bl_ref, o_ref):
    h = a_ref[...] + b_ref[...] + b2_ref[...]
    lg = jnp.dot(h, wl_ref[...], preferred_element_type=f32) + bl_ref[...]
    m = jnp.max(lg, axis=-1, keepdims=True)
    lse = m + jnp.log(jnp.sum(jnp.exp(lg - m), axis=-1, keepdims=True))
    o_ref[...] = lg - lse


_t3 = pl.pallas_call(
    _t3_body,
    grid=(NP // 1024,),
    in_specs=[pl.BlockSpec((1024, H), lambda i: (i, 0)),
              pl.BlockSpec((1024, H), lambda i: (i, 0)),
              pl.BlockSpec((1, H), lambda i: (0, 0)),
              pl.BlockSpec((H, 128), lambda i: (0, 0)),
              pl.BlockSpec((1, 128), lambda i: (0, 0))],
    out_specs=pl.BlockSpec((1024, 128), lambda i: (i, 0)),
    out_shape=jax.ShapeDtypeStruct((NP, 128), f32),
)


def kernel(x, edge_index, emb1, emb2, W1, b1, W2, b2, Wl, bl):
    src = edge_index[0].astype(i32)
    dst = edge_index[1].astype(i32)
    srcp = jnp.pad(src, (0, EP - E), constant_values=N).reshape(EPC, CH)
    dstp = jnp.pad(dst, (0, EP - E), constant_values=N).reshape(EPC, CH)
    e1f = jnp.pad(emb1, ((0, 8), (0, 0))).reshape(-1)
    e2f = jnp.pad(emb2, ((0, 8), (0, 0))).reshape(-1)
    zrows = jnp.zeros((NP, H), f32)

    seg1, seg2, gpk, c1p, c2p = _pass_a(srcp, dstp, e1f, e2f)

    xp = jnp.pad(x, ((0, NP - N), (0, 0)))
    W1b = W1.reshape(8, D, H).transpose(1, 0, 2).reshape(D, 8 * H)
    y = _t1(xp, W1b)
    y8 = y.reshape(YR, H)

    ic1m, ic2m = _tinv(c1p.reshape(NC, CNTN), c2p.reshape(NC, CNTN))
    ic1 = ic1m.reshape(CNTN)
    ic2 = ic2m.reshape(CNTN)

    h1p = _pass_b(gpk, dstp, seg1, seg2, ic1, ic2, y8, zrows
                  ).reshape(NC, NP, H)
    z = _t2(h1p[0], h1p[1], b1.reshape(1, H), W2 * 0.125)

    h2p = _pass_c(srcp, dstp, seg1, seg2, ic1, ic2, z, zrows
                  ).reshape(NC, NP, H)
    Wlp = jnp.pad(Wl, ((0, 0), (0, 128 - NCLS)))
    blp = jnp.pad(bl, (0, 128 - NCLS), constant_values=-1e30).reshape(1, 128)
    out = _t3(h2p[0], h2p[1], b2.reshape(1, H), Wlp, blp)
    return out[:N, :NCLS]


# speedup vs baseline: 9.5494x; 1.0821x over previous
import functools

import jax
import jax.numpy as jnp
from jax import lax
from jax.experimental import pallas as pl
from jax.experimental.pallas import tpu as pltpu
from jax.experimental.pallas import tpu_sc as plsc

f32 = jnp.float32
i32 = jnp.int32

N = 10000
D = 128
H = 64
NCLS = H // 2
NP = 10240
YR = NP * 8
E = 320000
NC, NS = 2, 16
NW = NC * NS
CH = 128
NCH = 80
EW = CH * NCH
EP = EW * NW
EPC = EP // CH
CNTN = 40960
CNT_PER_SUB = CNTN // NS
ROWS_PER_SUB = NP // NS

_mesh = plsc.VectorSubcoreMesh(core_axis_name="c", subcore_axis_name="s")
_params = pltpu.CompilerParams(needs_layout_passes=False,
                               use_tc_tiling_on_sc=False)


@functools.partial(
    pl.kernel,
    mesh=_mesh,
    compiler_params=_params,
    out_type=(
        jax.ShapeDtypeStruct((EPC, CH), i32),
        jax.ShapeDtypeStruct((EPC, CH), i32),
        jax.ShapeDtypeStruct((EPC, 2 * CH), i32),
        jax.ShapeDtypeStruct((NC * CNTN,), f32),
        jax.ShapeDtypeStruct((NC * CNTN,), f32),
    ),
    scratch_types=(
        pltpu.VMEM((2 * (N + 8),), f32),
        pltpu.VMEM((2 * (N + 8),), f32),
        pltpu.VMEM((CH,), i32),
        pltpu.VMEM((CH,), i32),
        pltpu.VMEM((CH,), i32),
        pltpu.VMEM((CH,), i32),
        pltpu.VMEM((CH,), i32),
        pltpu.VMEM((CH,), i32),
        pltpu.VMEM((CH,), i32),
        pltpu.VMEM((CH,), i32),
        pltpu.VMEM((CH,), f32),
        pltpu.VMEM((CH,), f32),
        pltpu.VMEM_SHARED((CNTN,), f32),
        pltpu.VMEM_SHARED((CNTN,), f32),
        pltpu.SemaphoreType.DMA,
        pltpu.SemaphoreType.DMA,
    ),
)
def _pass_a(srcp, dstp, e1f, e2f,
            seg1o, seg2o, gpo, c1o, c2o,
            e1v, e2v, src0, src1, dst0, dst1, s1v, s2v, g1v, g2v, onesv, zb,
            c1sh, c2sh, semi0, semi1):
    cid = lax.axis_index("c")
    sid = lax.axis_index("s")
    wid = sid * NC + cid

    srcb = (src0, src1)
    dstb = (dst0, dst1)
    semi = (semi0, semi1)

    pltpu.sync_copy(e1f, e1v)
    pltpu.sync_copy(e2f, e2v)
    for i in range(CH // 16):
        onesv[pl.ds(i * 16, 16)] = jnp.full((16,), 1.0, f32)
        zb[pl.ds(i * 16, 16)] = jnp.zeros((16,), f32)

    def zinit(k, carry):
        zsl = pl.ds(sid * CNT_PER_SUB + k * CH, CH)
        pltpu.sync_copy(zb, c1sh.at[zsl])
        pltpu.sync_copy(zb, c2sh.at[zsl])
        return carry

    lax.fori_loop(0, CNT_PER_SUB // CH, zinit, 0)
    plsc.subcore_barrier()

    def issue_in(c, b):
        row = wid * NCH + c
        pltpu.async_copy(srcp.at[row, :], srcb[b], semi[b])
        pltpu.async_copy(dstp.at[row, :], dstb[b], semi[b])

    def wait_in(c, b):
        row = wid * NCH + c
        pltpu.make_async_copy(srcp.at[row, :], srcb[b], semi[b]).wait()
        pltpu.make_async_copy(dstp.at[row, :], dstb[b], semi[b]).wait()

    issue_in(0, 0)
    issue_in(1, 1)

    def pair(p, carry):
        for b in (0, 1):
            c = 2 * p + b
            row = wid * NCH + c
            wait_in(c, b)
            two = jnp.full((16,), 2, i32)
            one = jnp.full((16,), 1, i32)
            zero = jnp.zeros((16,), i32)
            for i in range(CH // 16):
                sl = pl.ds(i * 16, 16)
                sv = srcb[b][sl]
                dv = dstb[b][sl]
                e1sx = plsc.load_gather(e1v, [sv * 2])
                e1sy = plsc.load_gather(e1v, [sv * 2 + 1])
                e1dx = plsc.load_gather(e1v, [dv * 2])
                e1dy = plsc.load_gather(e1v, [dv * 2 + 1])
                e2sx = plsc.load_gather(e2v, [sv * 2])
                e2sy = plsc.load_gather(e2v, [sv * 2 + 1])
                e2dx = plsc.load_gather(e2v, [dv * 2])
                e2dy = plsc.load_gather(e2v, [dv * 2 + 1])
                r1 = (jnp.where(e1dx - e1sx >= 0.0, two, zero)
                      + jnp.where(e1dy - e1sy >= 0.0, one, zero))
                r2 = (jnp.where(e2dx - e2sx >= 0.0, two, zero)
                      + jnp.where(e2dy - e2sy >= 0.0, one, zero))
                s1v[sl] = r1 * N + dv
                s2v[sl] = r2 * N + dv
                g1v[sl] = sv * 8 + r1
                g2v[sl] = sv * 8 + 4 + r2

            @pl.when(c + 2 < NCH)
            def _():
                issue_in(c + 2, b)

            pltpu.sync_copy(s1v, seg1o.at[row, :])
            pltpu.sync_copy(s2v, seg2o.at[row, :])
            pltpu.sync_copy(g1v, gpo.at[row, pl.ds(0, CH)])
            pltpu.sync_copy(g2v, gpo.at[row, pl.ds(CH, CH)])
            pltpu.sync_copy(onesv, c1sh.at[s1v], add=True)
            pltpu.sync_copy(onesv, c2sh.at[s2v], add=True)
        return carry

    lax.fori_loop(0, NCH // 2, pair, 0)
    plsc.subcore_barrier()

    def readout(k, carry):
        isl = pl.ds(sid * CNT_PER_SUB + k * CH, CH)
        osl = pl.ds(cid * CNTN + sid * CNT_PER_SUB + k * CH, CH)
        pltpu.sync_copy(c1sh.at[isl], zb)
        pltpu.sync_copy(zb, c1o.at[osl])
        pltpu.sync_copy(c2sh.at[isl], zb)
        pltpu.sync_copy(zb, c2o.at[osl])
        return carry

    lax.fori_loop(0, CNT_PER_SUB // CH, readout, 0)


@functools.partial(
    pl.kernel,
    mesh=_mesh,
    compiler_params=_params,
    out_type=jax.ShapeDtypeStruct((NC * NP, H), f32),
    scratch_types=(
        pltpu.VMEM((2 * CH,), i32),
        pltpu.VMEM((2 * CH,), i32),
        pltpu.VMEM((CH,), i32),
        pltpu.VMEM((CH,), i32),
        pltpu.VMEM((CH,), i32),
        pltpu.VMEM((CH,), i32),
        pltpu.VMEM((CH,), i32),
        pltpu.VMEM((CH,), i32),
        pltpu.VMEM((CH,), f32),
        pltpu.VMEM((CH,), f32),
        pltpu.VMEM((CH,), f32),
        pltpu.VMEM((CH,), f32),
        pltpu.VMEM((CH,), i32),
        pltpu.VMEM((CH,), i32),
        pltpu.VMEM((2 * CH, H), f32),
        pltpu.VMEM((2 * CH, H), f32),
        pltpu.VMEM((CH, H), f32),
        pltpu.VMEM((CH, H), f32),
        pltpu.VMEM_SHARED((NP, H), f32),
        pltpu.SemaphoreType.DMA,
        pltpu.SemaphoreType.DMA,
        pltpu.SemaphoreType.DMA,
        pltpu.SemaphoreType.DMA,
        pltpu.SemaphoreType.DMA,
        pltpu.SemaphoreType.DMA,
    ),
)
def _pass_b(gpi, dsti, s1i, s2i, ic1, ic2, y8, zrows,
            h1o,
            gp0, gp1, dst0, dst1, s10, s11, s20, s21, w10, w11, w20, w21,
            ds0, ds1, rw0, rw1, vb0, vb1,
            h1sh, semi0, semi1, semg0, semg1, sems0, sems1):
    cid = lax.axis_index("c")
    sid = lax.axis_index("s")
    wid = sid * NC + cid

    gp = (gp0, gp1)
    dstb = (dst0, dst1)
    s1b = (s10, s11)
    s2b = (s20, s21)
    w1b = (w10, w11)
    w2b = (w20, w21)
    dss = (ds0, ds1)
    rw = (rw0, rw1)
    vb = (vb0, vb1)
    semi = (semi0, semi1)
    semg = (semg0, semg1)
    sems = (sems0, sems1)

    def zinit(k, carry):
        rsl = pl.ds(sid * ROWS_PER_SUB + k * CH, CH)
        pltpu.sync_copy(zrows.at[rsl, :], vb0)
        pltpu.sync_copy(vb0, h1sh.at[rsl, :])
        return carry

    lax.fori_loop(0, ROWS_PER_SUB // CH, zinit, 0)
    plsc.subcore_barrier()

    def issue_idx(c, b):
        row = wid * NCH + c
        pltpu.async_copy(gpi.at[row, :], gp[b], semi[b])
        pltpu.async_copy(dsti.at[row, :], dstb[b], semi[b])
        pltpu.async_copy(s1i.at[row, :], s1b[b], semi[b])
        pltpu.async_copy(s2i.at[row, :], s2b[b], semi[b])

    def wait_idx(c, b):
        row = wid * NCH + c
        pltpu.make_async_copy(gpi.at[row, :], gp[b], semi[b]).wait()
        pltpu.make_async_copy(dsti.at[row, :], dstb[b], semi[b]).wait()
        pltpu.make_async_copy(s1i.at[row, :], s1b[b], semi[b]).wait()
        pltpu.make_async_copy(s2i.at[row, :], s2b[b], semi[b]).wait()

    def issue_gather(b):
        pltpu.async_copy(y8.at[gp[b]], rw[b], semg[b])
        pltpu.async_copy(ic1.at[s1b[b]], w1b[b], semg[b])
        pltpu.async_copy(ic2.at[s2b[b]], w2b[b], semg[b])

    def wait_gather(b):
        pltpu.make_async_copy(y8.at[gp[b]], rw[b], semg[b]).wait()
        pltpu.make_async_copy(ic1.at[s1b[b]], w1b[b], semg[b]).wait()
        pltpu.make_async_copy(ic2.at[s2b[b]], w2b[b], semg[b]).wait()

    def issue_scatter(b):
        pltpu.async_copy(vb[b], h1sh.at[dss[b]], sems[b], add=True)

    def wait_scatter(b):
        pltpu.make_async_copy(vb[b], h1sh.at[dss[b]], sems[b]).wait()

    def compute(b):
        for i in range(CH // 16):
            sl = pl.ds(i * 16, 16)
            dss[b][sl] = dstb[b][sl]

        def edge(p, cc):
            for u in range(8):
                j = p * 8 + u
                jv = jnp.full((16,), j, i32)
                bw1 = plsc.load_gather(w1b[b], [jv])
                bw2 = plsc.load_gather(w2b[b], [jv])
                for k in range(H // 16):
                    sl = pl.ds(k * 16, 16)
                    vb[b][j, sl] = (rw[b][j, sl] * bw1
                                    + rw[b][j + CH, sl] * bw2)
            return cc

        lax.fori_loop(0, CH // 8, edge, 0)

    issue_idx(0, 0)
    wait_idx(0, 0)
    issue_gather(0)
    issue_idx(1, 1)

    def pair(p, carry):
        for b in (0, 1):
            c = 2 * p + b
            bn = 1 - b
            wait_gather(b)

            @pl.when(c + 1 < NCH)
            def _():
                wait_idx(c + 1, bn)
                issue_gather(bn)

            @pl.when(c >= 2)
            def _():
                wait_scatter(b)

            compute(b)
            issue_scatter(b)

            @pl.when(c + 2 < NCH)
            def _():
                issue_idx(c + 2, b)
        return carry

    lax.fori_loop(0, NCH // 2, pair, 0)
    wait_scatter(0)
    wait_scatter(1)
    plsc.subcore_barrier()

    def readout(k, carry):
        isl = pl.ds(sid * ROWS_PER_SUB + k * CH, CH)
        osl = pl.ds(cid * NP + sid * ROWS_PER_SUB + k * CH, CH)
        pltpu.sync_copy(h1sh.at[isl, :], vb0)
        pltpu.sync_copy(vb0, h1o.at[osl, :])
        return carry

    lax.fori_loop(0, ROWS_PER_SUB // CH, readout, 0)


@functools.partial(
    pl.kernel,
    mesh=_mesh,
    compiler_params=_params,
    out_type=jax.ShapeDtypeStruct((NC * NP, H), f32),
    scratch_types=(
        pltpu.VMEM((CH,), i32),
        pltpu.VMEM((CH,), i32),
        pltpu.VMEM((CH,), i32),
        pltpu.VMEM((CH,), i32),
        pltpu.VMEM((CH,), i32),
        pltpu.VMEM((CH,), i32),
        pltpu.VMEM((CH,), i32),
        pltpu.VMEM((CH,), i32),
        pltpu.VMEM((CH,), f32),
        pltpu.VMEM((CH,), f32),
        pltpu.VMEM((CH,), f32),
        pltpu.VMEM((CH,), f32),
        pltpu.VMEM((CH,), i32),
        pltpu.VMEM((CH,), i32),
        pltpu.VMEM((CH, H), f32),
        pltpu.VMEM((CH, H), f32),
        pltpu.VMEM((CH, H), f32),
        pltpu.VMEM((CH, H), f32),
        pltpu.VMEM_SHARED((NP, H), f32),
        pltpu.SemaphoreType.DMA,
        pltpu.SemaphoreType.DMA,
        pltpu.SemaphoreType.DMA,
        pltpu.SemaphoreType.DMA,
        pltpu.SemaphoreType.DMA,
        pltpu.SemaphoreType.DMA,
    ),
)
def _pass_c(srcp, dstp, s1i, s2i, ic1, ic2, z, zrows,
            h2o,
            src0, src1, dst0, dst1, s10, s11, s20, s21, w10, w11, w20, w21,
            ds0, ds1, rw0, rw1, vb0, vb1,
            h2sh, semi0, semi1, semg0, semg1, sems0, sems1):
    cid = lax.axis_index("c")
    sid = lax.axis_index("s")
    wid = sid * NC + cid

    srcb = (src0, src1)
    dstb = (dst0, dst1)
    s1b = (s10, s11)
    s2b = (s20, s21)
    w1b = (w10, w11)
    w2b = (w20, w21)
    dss = (ds0, ds1)
    rw = (rw0, rw1)
    vb = (vb0, vb1)
    semi = (semi0, semi1)
    semg = (semg0, semg1)
    sems = (sems0, sems1)

    def zinit(k, carry):
        rsl = pl.ds(sid * ROWS_PER_SUB + k * CH, CH)
        pltpu.sync_copy(zrows.at[rsl, :], vb0)
        pltpu.sync_copy(vb0, h2sh.at[rsl, :])
        return carry

    lax.fori_loop(0, ROWS_PER_SUB // CH, zinit, 0)
    plsc.subcore_barrier()

    def issue_idx(c, b):
        row = wid * NCH + c
        pltpu.async_copy(srcp.at[row, :], srcb[b], semi[b])
        pltpu.async_copy(dstp.at[row, :], dstb[b], semi[b])
        pltpu.async_copy(s1i.at[row, :], s1b[b], semi[b])
        pltpu.async_copy(s2i.at[row, :], s2b[b], semi[b])

    def wait_idx(c, b):
        row = wid * NCH + c
        pltpu.make_async_copy(srcp.at[row, :], srcb[b], semi[b]).wait()
        pltpu.make_async_copy(dstp.at[row, :], dstb[b], semi[b]).wait()
        pltpu.make_async_copy(s1i.at[row, :], s1b[b], semi[b]).wait()
        pltpu.make_async_copy(s2i.at[row, :], s2b[b], semi[b]).wait()

    def issue_gather(b):
        pltpu.async_copy(z.at[srcb[b]], rw[b], semg[b])
        pltpu.async_copy(ic1.at[s1b[b]], w1b[b], semg[b])
        pltpu.async_copy(ic2.at[s2b[b]], w2b[b], semg[b])

    def wait_gather(b):
        pltpu.make_async_copy(z.at[srcb[b]], rw[b], semg[b]).wait()
        pltpu.make_async_copy(ic1.at[s1b[b]], w1b[b], semg[b]).wait()
        pltpu.make_async_copy(ic2.at[s2b[b]], w2b[b], semg[b]).wait()

    def issue_scatter(b):
        pltpu.async_copy(vb[b], h2sh.at[dss[b]], sems[b], add=True)

    def wait_scatter(b):
        pltpu.make_async_copy(vb[b], h2sh.at[dss[b]], sems[b]).wait()

    def compute(b):
        for i in range(CH // 16):
            sl = pl.ds(i * 16, 16)
            dss[b][sl] = dstb[b][sl]

        def edge(p, cc):
            for u in range(8):
                j = p * 8 + u
                jv = jnp.full((16,), j, i32)
                bw = (plsc.load_gather(w1b[b], [jv])
                      + plsc.load_gather(w2b[b], [jv]))
                for k in range(H // 16):
                    sl = pl.ds(k * 16, 16)
                    vb[b][j, sl] = rw[b][j, sl] * bw
            return cc

        lax.fori_loop(0, CH // 8, edge, 0)

    issue_idx(0, 0)
    wait_idx(0, 0)
    issue_gather(0)
    issue_idx(1, 1)

    def pair(p, carry):
        for b in (0, 1):
            c = 2 * p + b
            bn = 1 - b
            wait_gather(b)

            @pl.when(c + 1 < NCH)
            def _():
                wait_idx(c + 1, bn)
                issue_gather(bn)

            @pl.when(c >= 2)
            def _():
                wait_scatter(b)

            compute(b)
            issue_scatter(b)

            @pl.when(c + 2 < NCH)
            def _():
                issue_idx(c + 2, b)
        return carry

    lax.fori_loop(0, NCH // 2, pair, 0)
    wait_scatter(0)
    wait_scatter(1)
    plsc.subcore_barrier()

    def readout(k, carry):
        isl = pl.ds(sid * ROWS_PER_SUB + k * CH, CH)
        osl = pl.ds(cid * NP + sid * ROWS_PER_SUB + k * CH, CH)
        pltpu.sync_copy(h2sh.at[isl, :], vb0)
        pltpu.sync_copy(vb0, h2o.at[osl, :])
        return carry

    lax.fori_loop(0, ROWS_PER_SUB // CH, readout, 0)


def _t1_body(x_ref, w_ref, o_ref):
    o_ref[...] = jnp.dot(x_ref[...], w_ref[...], preferred_element_type=f32)


_t1 = pl.pallas_call(
    _t1_body,
    grid=(NP // 1024,),
    in_specs=[pl.BlockSpec((1024, D), lambda i: (i, 0)),
              pl.BlockSpec((D, 8 * H), lambda i: (0, 0))],
    out_specs=pl.BlockSpec((1024, 8 * H), lambda i: (i, 0)),
    out_shape=jax.ShapeDtypeStruct((NP, 8 * H), f32),
)


def _tinv_body(c1_ref, c2_ref, o1_ref, o2_ref):
    o1_ref[...] = 1.0 / jnp.maximum(
        jnp.sum(c1_ref[...], axis=0, keepdims=True), 1.0)
    o2_ref[...] = 1.0 / jnp.maximum(
        jnp.sum(c2_ref[...], axis=0, keepdims=True), 1.0)


_tinv = pl.pallas_call(
    _tinv_body,
    out_shape=(jax.ShapeDtypeStruct((1, CNTN), f32),
               jax.ShapeDtypeStruct((1, CNTN), f32)),
)


def _t2_body(a_ref, b_ref, b1_ref, w2_ref, o_ref):
    h = jnp.maximum(a_ref[...] + b_ref[...] + b1_ref[...], 0.0)
    o_ref[...] = jnp.dot(h, w2_ref[...], preferred_element_type=f32)


_t2 = pl.pallas_call(
    _t2_body,
    grid=(NP // 1024,),
    in_specs=[pl.BlockSpec((1024, H), lambda i: (i, 0)),
              pl.BlockSpec((1024, H), lambda i: (i, 0)),
              pl.BlockSpec((1, H), lambda i: (0, 0)),
              pl.BlockSpec((H, H), lambda i: (0, 0))],
    out_specs=pl.BlockSpec((1024, H), lambda i: (i, 0)),
    out_shape=jax.ShapeDtypeStruct((NP, H), f32),
)


def _t3_body(a_ref, b_ref, b2_ref, wl_ref, bl_ref, o_ref):
    h = a_ref[...] + b_ref[...] + b2_ref[...]
    lg = jnp.dot(h, wl_ref[...], preferred_element_type=f32) + bl_ref[...]
    m = jnp.max(lg, axis=-1, keepdims=True)
    lse = m + jnp.log(jnp.sum(jnp.exp(lg - m), axis=-1, keepdims=True))
    o_ref[...] = lg - lse


_t3 = pl.pallas_call(
    _t3_body,
    grid=(NP // 1024,),
    in_specs=[pl.BlockSpec((1024, H), lambda i: (i, 0)),
              pl.BlockSpec((1024, H), lambda i: (i, 0)),
              pl.BlockSpec((1, H), lambda i: (0, 0)),
              pl.BlockSpec((H, 128), lambda i: (0, 0)),
              pl.BlockSpec((1, 128), lambda i: (0, 0))],
    out_specs=pl.BlockSpec((1024, 128), lambda i: (i, 0)),
    out_shape=jax.ShapeDtypeStruct((NP, 128), f32),
)


def kernel(x, edge_index, emb1, emb2, W1, b1, W2, b2, Wl, bl):
    src = edge_index[0].astype(i32)
    dst = edge_index[1].astype(i32)
    srcp = jnp.pad(src, (0, EP - E), constant_values=N).reshape(EPC, CH)
    dstp = jnp.pad(dst, (0, EP - E), constant_values=N).reshape(EPC, CH)
    e1f = jnp.pad(emb1, ((0, 8), (0, 0))).reshape(-1)
    e2f = jnp.pad(emb2, ((0, 8), (0, 0))).reshape(-1)
    zrows = jnp.zeros((NP, H), f32)

    seg1, seg2, gpk, c1p, c2p = _pass_a(srcp, dstp, e1f, e2f)

    xp = jnp.pad(x, ((0, NP - N), (0, 0)))
    W1b = W1.reshape(8, D, H).transpose(1, 0, 2).reshape(D, 8 * H)
    y = _t1(xp, W1b)
    y8 = y.reshape(YR, H)

    ic1m, ic2m = _tinv(c1p.reshape(NC, CNTN), c2p.reshape(NC, CNTN))
    ic1 = ic1m.reshape(CNTN)
    ic2 = ic2m.reshape(CNTN)

    h1p = _pass_b(gpk, dstp, seg1, seg2, ic1, ic2, y8, zrows
                  ).reshape(NC, NP, H)
    z = _t2(h1p[0], h1p[1], b1.reshape(1, H), W2 * 0.125)

    h2p = _pass_c(srcp, dstp, seg1, seg2, ic1, ic2, z, zrows
                  ).reshape(NC, NP, H)
    Wlp = jnp.pad(Wl, ((0, 0), (0, 128 - NCLS)))
    blp = jnp.pad(bl, (0, 128 - NCLS), constant_values=-1e30).reshape(1, 128)
    out = _t3(h2p[0], h2p[1], b2.reshape(1, H), Wlp, blp)
    return out[:N, :NCLS]
